# baseline (speedup 1.0000x reference)
import functools

import jax
import jax.numpy as jnp
from jax import lax
from jax.experimental import pallas as pl
from jax.experimental.pallas import tpu as pltpu
from jax.experimental.pallas import tpu_sc as plsc

F32 = jnp.float32
I32 = jnp.int32
K = 64


def _fps_body(px_ref, py_ref, pz_ref, ox_ref, oy_ref, oz_ref, *, M):
    R, N2 = px_ref.shape
    B = R // 2
    px = px_ref[...]
    py = py_ref[...]
    pz = pz_ref[...]
    col = lax.broadcasted_iota(I32, (R, N2), 1)
    half = lax.broadcasted_iota(I32, (R, N2), 0) % 2
    gidx = col + half * N2
    even = lax.broadcasted_iota(I32, (R, 1), 0) % 2 == 0

    def pair_bcast(v8, comb):
        u = jnp.concatenate([v8[1:], v8[:1]], axis=0)
        d = jnp.concatenate([v8[-1:], v8[:-1]], axis=0)
        return comb(v8, jnp.where(even, u, d))

    col128 = lax.broadcasted_iota(I32, (R, 128), 1)
    zbuf = jnp.zeros((R, 128), F32)

    def inner(k, st):
        dists, bx, by, bz = st
        m8 = jnp.max(dists, axis=1, keepdims=True)
        m8b = pair_bcast(m8, jnp.maximum)
        cand = jnp.where(dists == m8b, gidx, 2 * N2)
        n8 = jnp.min(cand, axis=1, keepdims=True)
        n8b = pair_bcast(n8, jnp.minimum)
        selm = gidx == n8b
        qx8 = jnp.sum(jnp.where(selm, px, 0.0), axis=1, keepdims=True)
        qy8 = jnp.sum(jnp.where(selm, py, 0.0), axis=1, keepdims=True)
        qz8 = jnp.sum(jnp.where(selm, pz, 0.0), axis=1, keepdims=True)
        qx8b = pair_bcast(qx8, jnp.add)
        qy8b = pair_bcast(qy8, jnp.add)
        qz8b = pair_bcast(qz8, jnp.add)
        put = col128 == k
        bx = jnp.where(put, qx8b, bx)
        by = jnp.where(put, qy8b, by)
        bz = jnp.where(put, qz8b, bz)
        ddx = px - qx8b
        ddy = py - qy8b
        ddz = pz - qz8b
        d = (ddx * ddx + ddy * ddy) + ddz * ddz
        return (jnp.minimum(dists, d), bx, by, bz)

    def outer(j, dists):
        dists, bx, by, bz = lax.fori_loop(
            0, 128, inner, (dists, zbuf, zbuf, zbuf))
        ox_ref[:, pl.ds(j * 128, 128)] = bx
        oy_ref[:, pl.ds(j * 128, 128)] = by
        oz_ref[:, pl.ds(j * 128, 128)] = bz
        return dists

    lax.fori_loop(0, M // 128, outer, jnp.full((R, N2), 3.0e38, F32))


def _fps(px, py, pz, M):
    R = px.shape[0]
    out = jax.ShapeDtypeStruct((R, M), F32)
    ox, oy, oz = pl.pallas_call(
        functools.partial(_fps_body, M=M),
        out_shape=[out, out, out],
    )(px, py, pz)
    return ox[0::2], oy[0::2], oz[0::2]


def _iota16():
    return lax.broadcasted_iota(I32, (16,), 0)


def _splat(val, dtype=I32):
    return jnp.full((16,), val, dtype)


def _sc_search_gather_sa1(px, py, pz, qx, qy, qz, r2):
    B, N = px.shape
    BQ = qx.shape[0]
    Q = BQ // B
    NW = 32
    QPW = BQ // NW
    WPB = NW // B
    mesh = plsc.VectorSubcoreMesh(core_axis_name="c", subcore_axis_name="s")

    @functools.partial(
        pl.kernel,
        mesh=mesh,
        compiler_params=pltpu.CompilerParams(needs_layout_passes=False),
        out_type=[
            jax.ShapeDtypeStruct((BQ * K * 8,), F32),
            jax.ShapeDtypeStruct((BQ,), I32),
        ],
        scratch_types=[
            pltpu.VMEM((N,), F32),
            pltpu.VMEM((N,), F32),
            pltpu.VMEM((N,), F32),
            pltpu.VMEM((QPW,), F32),
            pltpu.VMEM((QPW,), F32),
            pltpu.VMEM((QPW,), F32),
            pltpu.VMEM((K * 8,), F32),
            pltpu.VMEM((K * 8,), F32),
            pltpu.VMEM((QPW,), I32),
        ],
    )
    def body(px_h, py_h, pz_h, qx_h, qy_h, qz_h, feat_h, cnt_h,
             pxs, pys, pzs, qxs, qys, qzs, ot0, ot1, cnts):
        wid = lax.axis_index("s") * 2 + lax.axis_index("c")
        b = wid // WPB
        qbase = wid * QPW
        pltpu.sync_copy(px_h.at[b], pxs)
        pltpu.sync_copy(py_h.at[b], pys)
        pltpu.sync_copy(pz_h.at[b], pzs)
        pltpu.sync_copy(qx_h.at[pl.ds(qbase, QPW)], qxs)
        pltpu.sync_copy(qy_h.at[pl.ds(qbase, QPW)], qys)
        pltpu.sync_copy(qz_h.at[pl.ds(qbase, QPW)], qzs)

        zf = jnp.zeros((16,), F32)
        for j in range(K * 8 // 16):
            ot0[pl.ds(j * 16, 16)] = zf
            ot1[pl.ds(j * 16, 16)] = zf

        lanes = _iota16()

        def per_pair(p, _):
            q0 = p * 2
            iq0 = _splat(q0)
            iq1 = iq0 + 1
            qxv0 = plsc.load_gather(qxs, [iq0])
            qyv0 = plsc.load_gather(qys, [iq0])
            qzv0 = plsc.load_gather(qzs, [iq0])
            qxv1 = plsc.load_gather(qxs, [iq1])
            qyv1 = plsc.load_gather(qys, [iq1])
            qzv1 = plsc.load_gather(qzs, [iq1])

            def chunk(ci, carry):
                cnt0, cnt1 = carry
                base = ci * 16
                pxv = pxs[pl.ds(base, 16)]
                pyv = pys[pl.ds(base, 16)]
                pzv = pzs[pl.ds(base, 16)]
                dx0 = pxv - qxv0
                dy0 = pyv - qyv0
                dz0 = pzv - qzv0
                dx1 = pxv - qxv1
                dy1 = pyv - qyv1
                dz1 = pzv - qzv1
                d20 = (dx0 * dx0 + dy0 * dy0) + dz0 * dz0
                d21 = (dx1 * dx1 + dy1 * dy1) + dz1 * dz1
                m0 = d20 <= r2
                m1 = d21 <= r2
                s0 = cnt0 + plsc.cumsum(m0.astype(I32)) - 1
                s1 = cnt1 + plsc.cumsum(m1.astype(I32)) - 1
                ok0 = m0 & (s0 < K)
                ok1 = m1 & (s1 < K)
                f0 = s0 * 8
                f1 = s1 * 8
                plsc.store_scatter(ot0, [f0], pxv, mask=ok0)
                plsc.store_scatter(ot1, [f1], pxv, mask=ok1)
                plsc.store_scatter(ot0, [f0 + 1], pyv, mask=ok0)
                plsc.store_scatter(ot1, [f1 + 1], pyv, mask=ok1)
                plsc.store_scatter(ot0, [f0 + 2], pzv, mask=ok0)
                plsc.store_scatter(ot1, [f1 + 2], pzv, mask=ok1)
                plsc.store_scatter(ot0, [f0 + 3], dx0, mask=ok0)
                plsc.store_scatter(ot1, [f1 + 3], dx1, mask=ok1)
                plsc.store_scatter(ot0, [f0 + 4], dy0, mask=ok0)
                plsc.store_scatter(ot1, [f1 + 4], dy1, mask=ok1)
                plsc.store_scatter(ot0, [f0 + 5], dz0, mask=ok0)
                plsc.store_scatter(ot1, [f1 + 5], dz1, mask=ok1)
                return (cnt0 + plsc.all_reduce_population_count(m0),
                        cnt1 + plsc.all_reduce_population_count(m1))

            z16 = jnp.zeros((16,), I32)
            cnt0, cnt1 = lax.fori_loop(0, N // 16, chunk, (z16, z16))
            plsc.store_scatter(cnts, [iq0], jnp.minimum(cnt0, K),
                               mask=lanes == 0)
            plsc.store_scatter(cnts, [iq1], jnp.minimum(cnt1, K),
                               mask=lanes == 0)
            pltpu.sync_copy(ot0, feat_h.at[pl.ds((qbase + q0) * K * 8,
                                                 K * 8)])
            pltpu.sync_copy(ot1, feat_h.at[pl.ds((qbase + q0 + 1) * K * 8,
                                                 K * 8)])
            return 0

        lax.fori_loop(0, QPW // 2, per_pair, 0)
        pltpu.sync_copy(cnts, cnt_h.at[pl.ds(qbase, QPW)])

    return body(px, py, pz, qx, qy, qz)


def _sc_search_gather_sa2(px, py, pz, qx, qy, qz, t1, r2):
    B, N = px.shape
    BQ = qx.shape[0]
    NW = 32
    QPW = BQ // NW
    WPB = NW // B
    D = t1.shape[1]
    mesh = plsc.VectorSubcoreMesh(core_axis_name="c", subcore_axis_name="s")

    @functools.partial(
        pl.kernel,
        mesh=mesh,
        compiler_params=pltpu.CompilerParams(needs_layout_passes=False),
        out_type=[
            jax.ShapeDtypeStruct((BQ, K, D), F32),
            jax.ShapeDtypeStruct((BQ,), I32),
        ],
        scratch_types=[
            pltpu.VMEM((N,), F32),
            pltpu.VMEM((N,), F32),
            pltpu.VMEM((N,), F32),
            pltpu.VMEM((QPW,), F32),
            pltpu.VMEM((QPW,), F32),
            pltpu.VMEM((QPW,), F32),
            pltpu.VMEM((2, K), I32),
            pltpu.VMEM((2, K, D), F32),
            pltpu.VMEM((QPW,), I32),
            pltpu.SemaphoreType.DMA,
            pltpu.SemaphoreType.DMA,
            pltpu.SemaphoreType.DMA,
            pltpu.SemaphoreType.DMA,
        ],
    )
    def body(px_h, py_h, pz_h, qx_h, qy_h, qz_h, t1_h, g_h, cnt_h,
             pxs, pys, pzs, qxs, qys, qzs, idxb, rows, cnts,
             sem0, sem1, osem0, osem1):
        wid = lax.axis_index("s") * 2 + lax.axis_index("c")
        b = wid // WPB
        qbase = wid * QPW
        pbase = b * N
        pltpu.sync_copy(px_h.at[b], pxs)
        pltpu.sync_copy(py_h.at[b], pys)
        pltpu.sync_copy(pz_h.at[b], pzs)
        pltpu.sync_copy(qx_h.at[pl.ds(qbase, QPW)], qxs)
        pltpu.sync_copy(qy_h.at[pl.ds(qbase, QPW)], qys)
        pltpu.sync_copy(qz_h.at[pl.ds(qbase, QPW)], qzs)

        zi = jnp.zeros((16,), I32)
        for buf in range(2):
            for j in range(K // 16):
                idxb.at[buf][pl.ds(j * 16, 16)] = zi + pbase

        lanes = _iota16()
        sems = (sem0, sem1)
        osems = (osem0, osem1)

        def scan_query(q, idx_ref):
            iq = _splat(q)
            qxv = plsc.load_gather(qxs, [iq])
            qyv = plsc.load_gather(qys, [iq])
            qzv = plsc.load_gather(qzs, [iq])

            def chunk(ci, cnt_v):
                base = ci * 16
                pxv = pxs[pl.ds(base, 16)]
                pyv = pys[pl.ds(base, 16)]
                pzv = pzs[pl.ds(base, 16)]
                dx = pxv - qxv
                dy = pyv - qyv
                dz = pzv - qzv
                d2 = (dx * dx + dy * dy) + dz * dz
                m = d2 <= r2
                slot = cnt_v + plsc.cumsum(m.astype(I32)) - 1
                ok = m & (slot < K)
                plsc.store_scatter(idx_ref, [slot],
                                   lanes + (base + pbase), mask=ok)
                return cnt_v + plsc.all_reduce_population_count(m)

            cnt_v = lax.fori_loop(0, N // 16, chunk, jnp.zeros((16,), I32))
            plsc.store_scatter(cnts, [iq], jnp.minimum(cnt_v, K),
                               mask=lanes == 0)

        def per_pair(p, _):
            for buf in range(2):
                q = p * 2 + buf
                idx_ref = idxb.at[buf]
                row_ref = rows.at[buf]

                @pl.when(p > 0)
                def _():
                    pltpu.make_async_copy(
                        t1_h.at[idx_ref], row_ref, sems[buf]).wait()
                    pltpu.async_copy(row_ref, g_h.at[qbase + q - 2],
                                     osems[buf])

                scan_query(q, idx_ref)

                @pl.when(p > 0)
                def _():
                    pltpu.make_async_copy(
                        row_ref, g_h.at[qbase], osems[buf]).wait()

                pltpu.async_copy(t1_h.at[idx_ref], row_ref, sems[buf])
            return 0

        lax.fori_loop(0, QPW // 2, per_pair, 0)
        for buf in range(2):
            pltpu.make_async_copy(
                t1_h.at[idxb.at[buf]], rows.at[buf], sems[buf]).wait()
            pltpu.sync_copy(rows.at[buf], g_h.at[qbase + QPW - 2 + buf])
        pltpu.sync_copy(cnts, cnt_h.at[pl.ds(qbase, QPW)])

    return body(px, py, pz, qx, qy, qz, t1)


def _mlp1_body(feat_ref, pen_ref, w0_ref, b0_ref, w1_ref, b1_ref, w2_ref,
               b2_ref, pos1_ref, wx2_ref, wr2_ref, x1_ref, t1_ref):
    QB = feat_ref.shape[0] // K
    X = feat_ref[...]
    h = jnp.dot(X, w0_ref[...], preferred_element_type=F32) + b0_ref[0:1, :]
    h = jnp.maximum(h, 0.0)
    h = jnp.dot(h, w1_ref[...], preferred_element_type=F32) + b1_ref[0:1, :]
    h = jnp.maximum(h, 0.0)
    h = jnp.dot(h, w2_ref[...], preferred_element_type=F32) + b2_ref[0:1, :]
    h = h + pen_ref[...]
    x1 = jnp.max(h.reshape(QB, K, 128), axis=1)
    x1_ref[...] = x1
    t1_ref[...] = (
        jnp.dot(x1, wx2_ref[...], preferred_element_type=F32)
        + jnp.dot(pos1_ref[...], wr2_ref[...], preferred_element_type=F32))


def _mlp2_body(g_ref, pen_ref, pos2_ref, wr2_ref, b0_ref, w1_ref, b1_ref,
               w2_ref, b2_ref, x2_ref):
    QB, _, D = g_ref.shape
    U = jnp.dot(pos2_ref[...], wr2_ref[...], preferred_element_type=F32)
    h = (g_ref[...]
         - lax.broadcast_in_dim(U, (QB, K, D), (0, 2))
         + lax.broadcast_in_dim(b0_ref[0:1, :], (QB, K, D), (1, 2)))
    h = jnp.maximum(h, 0.0).reshape(QB * K, D)
    h = jnp.dot(h, w1_ref[...], preferred_element_type=F32) + b1_ref[0:1, :]
    h = jnp.maximum(h, 0.0)
    h = jnp.dot(h, w2_ref[...], preferred_element_type=F32) + b2_ref[0:1, :]
    h = h + pen_ref[...]
    x2_ref[...] = jnp.max(h.reshape(QB, K, 256), axis=1)


def _stage3_body(x2_ref, pos2_ref, w3a_ref, w3b_ref, b0_ref, w1_ref, b1_ref,
                 w2_ref, b2_ref, wfc_ref, bfc_ref, out_ref):
    x2 = x2_ref[0]
    p = pos2_ref[0]
    h = (jnp.dot(x2, w3a_ref[...], preferred_element_type=F32)
         + jnp.dot(p, w3b_ref[...], preferred_element_type=F32)
         + b0_ref[0:1, :])
    h = jnp.maximum(h, 0.0)
    h = jnp.dot(h, w1_ref[...], preferred_element_type=F32) + b1_ref[0:1, :]
    h = jnp.maximum(h, 0.0)
    h = jnp.dot(h, w2_ref[...], preferred_element_type=F32) + b2_ref[0:1, :]
    g = jnp.max(h, axis=0, keepdims=True)
    out_ref[0] = jnp.dot(g, wfc_ref[...], preferred_element_type=F32) \
        + bfc_ref[0:1, :]


def _tile8(b):
    return jnp.tile(b.reshape(1, -1), (8, 1))


def _penalty(cnt):
    BQ = cnt.shape[0]
    kio = lax.broadcasted_iota(I32, (BQ, K), 1)
    pen = jnp.where(kio < cnt[:, None], 0.0, -jnp.inf).astype(F32)
    return pen.reshape(BQ * K, 1)


def _mlp1(feat, cnt, pos1p, W0p, b0, W1, b1, W2, b2, Wx2, Wr2p):
    BQ = cnt.shape[0]
    QB = 64
    grid = BQ // QB
    feat = feat.reshape(BQ * K, 8)
    full = lambda shp: pl.BlockSpec(shp, lambda i: (0, 0))
    return pl.pallas_call(
        _mlp1_body,
        grid=(grid,),
        in_specs=[
            pl.BlockSpec((QB * K, 8), lambda i: (i, 0)),
            pl.BlockSpec((QB * K, 1), lambda i: (i, 0)),
            full((8, 64)), full((8, 64)),
            full((64, 64)), full((8, 64)),
            full((64, 128)), full((8, 128)),
            pl.BlockSpec((QB, 8), lambda i: (i, 0)),
            full((128, 128)), full((8, 128)),
        ],
        out_specs=[
            pl.BlockSpec((QB, 128), lambda i: (i, 0)),
            pl.BlockSpec((QB, 128), lambda i: (i, 0)),
        ],
        out_shape=[
            jax.ShapeDtypeStruct((BQ, 128), F32),
            jax.ShapeDtypeStruct((BQ, 128), F32),
        ],
    )(feat, _penalty(cnt), W0p, _tile8(b0), W1, _tile8(b1), W2, _tile8(b2),
      pos1p, Wx2, Wr2p)


def _mlp2(G, cnt, pos2p, Wr2p, b0, W1, b1, W2, b2):
    BQ = G.shape[0]
    QB = 32
    grid = BQ // QB
    full = lambda shp: pl.BlockSpec(shp, lambda i: (0, 0))
    return pl.pallas_call(
        _mlp2_body,
        grid=(grid,),
        in_specs=[
            pl.BlockSpec((QB, K, 128), lambda i: (i, 0, 0)),
            pl.BlockSpec((QB * K, 1), lambda i: (i, 0)),
            pl.BlockSpec((QB, 8), lambda i: (i, 0)),
            full((8, 128)), full((8, 128)),
            full((128, 128)), full((8, 128)),
            full((128, 256)), full((8, 256)),
        ],
        out_specs=pl.BlockSpec((QB, 256), lambda i: (i, 0)),
        out_shape=jax.ShapeDtypeStruct((BQ, 256), F32),
    )(G, _penalty(cnt), pos2p, Wr2p, _tile8(b0), W1, _tile8(b1), W2,
      _tile8(b2))


def _stage3(x2, pos2p, W3a, W3bp, b0, W1, b1, W2, b2, Wfc, bfc):
    B = x2.shape[0]
    full = lambda shp: pl.BlockSpec(shp, lambda i: (0, 0))
    out = pl.pallas_call(
        _stage3_body,
        grid=(B,),
        in_specs=[
            pl.BlockSpec((1, 512, 256), lambda i: (i, 0, 0)),
            pl.BlockSpec((1, 512, 8), lambda i: (i, 0, 0)),
            full((256, 256)), full((8, 256)), full((8, 256)),
            full((256, 512)), full((8, 512)),
            full((512, 1024)), full((8, 1024)),
            full((1024, 256)), full((8, 256)),
        ],
        out_specs=pl.BlockSpec((1, 1, 256), lambda i: (i, 0, 0)),
        out_shape=jax.ShapeDtypeStruct((B, 1, 256), F32),
    )(x2, pos2p, W3a, W3bp, _tile8(b0), W1, _tile8(b1), W2, _tile8(b2),
      Wfc, _tile8(bfc))
    return out[:, 0, :]


def kernel(data, W1_0, b1_0, W1_1, b1_1, W1_2, b1_2, W2_0, b2_0, W2_1, b2_1,
           W2_2, b2_2, W3_0, b3_0, W3_1, b3_1, W3_2, b3_2, Wfc, bfc):
    B, N, _ = data.shape
    Q1, Q2 = N // 2, N // 8

    p0 = [data[:, :, c] for c in range(3)]
    p1 = _fps(*[c.reshape(2 * B, N // 2) for c in p0], Q1)
    p2 = _fps(*[c.reshape(2 * B, Q1 // 2) for c in p1], Q2)

    q1 = [c.reshape(B * Q1) for c in p1]
    q2 = [c.reshape(B * Q2) for c in p2]

    feat1, cnt1 = _sc_search_gather_sa1(*p0, *q1, F32(0.2 * 0.2))

    pos1p = jnp.pad(jnp.stack(p1, axis=-1).reshape(B * Q1, 3),
                    ((0, 0), (0, 5)))
    pos2p = jnp.pad(jnp.stack(p2, axis=-1).reshape(B * Q2, 3),
                    ((0, 0), (0, 5)))
    W0p = jnp.concatenate([W1_0, jnp.zeros((2, 64), F32)], axis=0)
    Wx2 = W2_0[:128]
    Wr2p = jnp.concatenate([W2_0[128:], jnp.zeros((5, 128), F32)], axis=0)

    x1, T1 = _mlp1(feat1, cnt1, pos1p, W0p, b1_0, W1_1, b1_1, W1_2, b1_2,
                   Wx2, Wr2p)

    G, cnt2 = _sc_search_gather_sa2(*p1, *q2, T1, F32(0.4 * 0.4))
    x2 = _mlp2(G, cnt2, pos2p, Wr2p, b2_0, W2_1, b2_1, W2_2, b2_2)

    W3a = W3_0[:256]
    W3bp = jnp.concatenate([W3_0[256:], jnp.zeros((5, 256), F32)], axis=0)
    out = _stage3(x2.reshape(B, Q2, 256), pos2p.reshape(B, Q2, 8),
                  W3a, W3bp, b3_0, W3_1, b3_1, W3_2, b3_2, Wfc, bfc)
    return out


# speedup vs baseline: 1.0704x; 1.0704x over previous
import functools

import jax
import jax.numpy as jnp
from jax import lax
from jax.experimental import pallas as pl
from jax.experimental.pallas import tpu as pltpu
from jax.experimental.pallas import tpu_sc as plsc

F32 = jnp.float32
I32 = jnp.int32
K = 64


def _fps_body(px_ref, py_ref, pz_ref, ox_ref, oy_ref, oz_ref, *, M):
    B, N = px_ref.shape
    px = px_ref[...]
    py = py_ref[...]
    pz = pz_ref[...]
    iota_n = lax.broadcasted_iota(I32, (B, N), 1)
    iota_m = lax.broadcasted_iota(I32, (B, M), 1)

    def body(i, st):
        dists, sx, sy, sz = st
        m = jnp.max(dists, axis=1, keepdims=True)
        nxt = jnp.min(jnp.where(dists == m, iota_n, N), axis=1,
                      keepdims=True)
        selm = iota_n == nxt
        qx = jnp.sum(jnp.where(selm, px, 0.0), axis=1, keepdims=True)
        qy = jnp.sum(jnp.where(selm, py, 0.0), axis=1, keepdims=True)
        qz = jnp.sum(jnp.where(selm, pz, 0.0), axis=1, keepdims=True)
        ddx = px - qx
        ddy = py - qy
        ddz = pz - qz
        d = (ddx * ddx + ddy * ddy) + ddz * ddz
        put = iota_m == i
        return (jnp.minimum(dists, d), jnp.where(put, qx, sx),
                jnp.where(put, qy, sy), jnp.where(put, qz, sz))

    z = jnp.zeros((B, M), F32)
    _, sx, sy, sz = lax.fori_loop(
        0, M, body, (jnp.full((B, N), 3.0e38, F32), z, z, z))
    ox_ref[...] = sx
    oy_ref[...] = sy
    oz_ref[...] = sz


def _fps(px, py, pz, M):
    B = px.shape[0]
    out = jax.ShapeDtypeStruct((B, M), F32)
    return pl.pallas_call(
        functools.partial(_fps_body, M=M),
        out_shape=[out, out, out],
    )(px, py, pz)


def _iota16():
    return lax.broadcasted_iota(I32, (16,), 0)


def _splat(val, dtype=I32):
    return jnp.full((16,), val, dtype)


def _sc_search_gather_sa1(px, py, pz, qx, qy, qz, r2):
    B, N = px.shape
    BQ = qx.shape[0]
    Q = BQ // B
    NW = 32
    QPW = BQ // NW
    WPB = NW // B
    mesh = plsc.VectorSubcoreMesh(core_axis_name="c", subcore_axis_name="s")

    @functools.partial(
        pl.kernel,
        mesh=mesh,
        compiler_params=pltpu.CompilerParams(needs_layout_passes=False),
        out_type=[
            jax.ShapeDtypeStruct((BQ * K * 8,), F32),
            jax.ShapeDtypeStruct((BQ,), I32),
        ],
        scratch_types=[
            pltpu.VMEM((N,), F32),
            pltpu.VMEM((N,), F32),
            pltpu.VMEM((N,), F32),
            pltpu.VMEM((QPW,), F32),
            pltpu.VMEM((QPW,), F32),
            pltpu.VMEM((QPW,), F32),
            pltpu.VMEM((K * 8,), F32),
            pltpu.VMEM((K * 8,), F32),
            pltpu.VMEM((QPW,), I32),
        ],
    )
    def body(px_h, py_h, pz_h, qx_h, qy_h, qz_h, feat_h, cnt_h,
             pxs, pys, pzs, qxs, qys, qzs, ot0, ot1, cnts):
        wid = lax.axis_index("s") * 2 + lax.axis_index("c")
        b = wid // WPB
        qbase = wid * QPW
        pltpu.sync_copy(px_h.at[b], pxs)
        pltpu.sync_copy(py_h.at[b], pys)
        pltpu.sync_copy(pz_h.at[b], pzs)
        pltpu.sync_copy(qx_h.at[pl.ds(qbase, QPW)], qxs)
        pltpu.sync_copy(qy_h.at[pl.ds(qbase, QPW)], qys)
        pltpu.sync_copy(qz_h.at[pl.ds(qbase, QPW)], qzs)

        zf = jnp.zeros((16,), F32)
        for j in range(K * 8 // 16):
            ot0[pl.ds(j * 16, 16)] = zf
            ot1[pl.ds(j * 16, 16)] = zf

        lanes = _iota16()

        def per_pair(p, _):
            q0 = p * 2
            iq0 = _splat(q0)
            iq1 = iq0 + 1
            qxv0 = plsc.load_gather(qxs, [iq0])
            qyv0 = plsc.load_gather(qys, [iq0])
            qzv0 = plsc.load_gather(qzs, [iq0])
            qxv1 = plsc.load_gather(qxs, [iq1])
            qyv1 = plsc.load_gather(qys, [iq1])
            qzv1 = plsc.load_gather(qzs, [iq1])

            def chunk(ci, carry):
                cnt0, cnt1 = carry
                base = ci * 16
                pxv = pxs[pl.ds(base, 16)]
                pyv = pys[pl.ds(base, 16)]
                pzv = pzs[pl.ds(base, 16)]
                dx0 = pxv - qxv0
                dy0 = pyv - qyv0
                dz0 = pzv - qzv0
                dx1 = pxv - qxv1
                dy1 = pyv - qyv1
                dz1 = pzv - qzv1
                d20 = (dx0 * dx0 + dy0 * dy0) + dz0 * dz0
                d21 = (dx1 * dx1 + dy1 * dy1) + dz1 * dz1
                m0 = d20 <= r2
                m1 = d21 <= r2
                s0 = cnt0 + plsc.cumsum(m0.astype(I32)) - 1
                s1 = cnt1 + plsc.cumsum(m1.astype(I32)) - 1
                ok0 = m0 & (s0 < K)
                ok1 = m1 & (s1 < K)
                f0 = s0 * 8
                f1 = s1 * 8
                plsc.store_scatter(ot0, [f0], pxv, mask=ok0)
                plsc.store_scatter(ot1, [f1], pxv, mask=ok1)
                plsc.store_scatter(ot0, [f0 + 1], pyv, mask=ok0)
                plsc.store_scatter(ot1, [f1 + 1], pyv, mask=ok1)
                plsc.store_scatter(ot0, [f0 + 2], pzv, mask=ok0)
                plsc.store_scatter(ot1, [f1 + 2], pzv, mask=ok1)
                plsc.store_scatter(ot0, [f0 + 3], dx0, mask=ok0)
                plsc.store_scatter(ot1, [f1 + 3], dx1, mask=ok1)
                plsc.store_scatter(ot0, [f0 + 4], dy0, mask=ok0)
                plsc.store_scatter(ot1, [f1 + 4], dy1, mask=ok1)
                plsc.store_scatter(ot0, [f0 + 5], dz0, mask=ok0)
                plsc.store_scatter(ot1, [f1 + 5], dz1, mask=ok1)
                return (cnt0 + plsc.all_reduce_population_count(m0),
                        cnt1 + plsc.all_reduce_population_count(m1))

            z16 = jnp.zeros((16,), I32)
            cnt0, cnt1 = lax.fori_loop(0, N // 16, chunk, (z16, z16))
            plsc.store_scatter(cnts, [iq0], jnp.minimum(cnt0, K),
                               mask=lanes == 0)
            plsc.store_scatter(cnts, [iq1], jnp.minimum(cnt1, K),
                               mask=lanes == 0)
            pltpu.sync_copy(ot0, feat_h.at[pl.ds((qbase + q0) * K * 8,
                                                 K * 8)])
            pltpu.sync_copy(ot1, feat_h.at[pl.ds((qbase + q0 + 1) * K * 8,
                                                 K * 8)])
            return 0

        lax.fori_loop(0, QPW // 2, per_pair, 0)
        pltpu.sync_copy(cnts, cnt_h.at[pl.ds(qbase, QPW)])

    return body(px, py, pz, qx, qy, qz)


def _sc_search_gather_sa2(px, py, pz, qx, qy, qz, t1, r2):
    B, N = px.shape
    BQ = qx.shape[0]
    NW = 32
    QPW = BQ // NW
    WPB = NW // B
    D = t1.shape[1]
    mesh = plsc.VectorSubcoreMesh(core_axis_name="c", subcore_axis_name="s")

    @functools.partial(
        pl.kernel,
        mesh=mesh,
        compiler_params=pltpu.CompilerParams(needs_layout_passes=False),
        out_type=[
            jax.ShapeDtypeStruct((BQ, K, D), F32),
            jax.ShapeDtypeStruct((BQ,), I32),
        ],
        scratch_types=[
            pltpu.VMEM((N,), F32),
            pltpu.VMEM((N,), F32),
            pltpu.VMEM((N,), F32),
            pltpu.VMEM((QPW,), F32),
            pltpu.VMEM((QPW,), F32),
            pltpu.VMEM((QPW,), F32),
            pltpu.VMEM((2, K), I32),
            pltpu.VMEM((2, K, D), F32),
            pltpu.VMEM((QPW,), I32),
            pltpu.SemaphoreType.DMA,
            pltpu.SemaphoreType.DMA,
            pltpu.SemaphoreType.DMA,
            pltpu.SemaphoreType.DMA,
        ],
    )
    def body(px_h, py_h, pz_h, qx_h, qy_h, qz_h, t1_h, g_h, cnt_h,
             pxs, pys, pzs, qxs, qys, qzs, idxb, rows, cnts,
             sem0, sem1, osem0, osem1):
        wid = lax.axis_index("s") * 2 + lax.axis_index("c")
        b = wid // WPB
        qbase = wid * QPW
        pbase = b * N
        pltpu.sync_copy(px_h.at[b], pxs)
        pltpu.sync_copy(py_h.at[b], pys)
        pltpu.sync_copy(pz_h.at[b], pzs)
        pltpu.sync_copy(qx_h.at[pl.ds(qbase, QPW)], qxs)
        pltpu.sync_copy(qy_h.at[pl.ds(qbase, QPW)], qys)
        pltpu.sync_copy(qz_h.at[pl.ds(qbase, QPW)], qzs)

        zi = jnp.zeros((16,), I32)
        for buf in range(2):
            for j in range(K // 16):
                idxb.at[buf][pl.ds(j * 16, 16)] = zi + pbase

        lanes = _iota16()
        sems = (sem0, sem1)
        osems = (osem0, osem1)

        def scan_query(q, idx_ref):
            iq = _splat(q)
            qxv = plsc.load_gather(qxs, [iq])
            qyv = plsc.load_gather(qys, [iq])
            qzv = plsc.load_gather(qzs, [iq])

            def chunk(ci, cnt_v):
                base = ci * 16
                pxv = pxs[pl.ds(base, 16)]
                pyv = pys[pl.ds(base, 16)]
                pzv = pzs[pl.ds(base, 16)]
                dx = pxv - qxv
                dy = pyv - qyv
                dz = pzv - qzv
                d2 = (dx * dx + dy * dy) + dz * dz
                m = d2 <= r2
                slot = cnt_v + plsc.cumsum(m.astype(I32)) - 1
                ok = m & (slot < K)
                plsc.store_scatter(idx_ref, [slot],
                                   lanes + (base + pbase), mask=ok)
                return cnt_v + plsc.all_reduce_population_count(m)

            cnt_v = lax.fori_loop(0, N // 16, chunk, jnp.zeros((16,), I32))
            plsc.store_scatter(cnts, [iq], jnp.minimum(cnt_v, K),
                               mask=lanes == 0)

        def per_pair(p, _):
            for buf in range(2):
                q = p * 2 + buf
                idx_ref = idxb.at[buf]
                row_ref = rows.at[buf]

                @pl.when(p > 0)
                def _():
                    pltpu.make_async_copy(
                        t1_h.at[idx_ref], row_ref, sems[buf]).wait()
                    pltpu.async_copy(row_ref, g_h.at[qbase + q - 2],
                                     osems[buf])

                scan_query(q, idx_ref)

                @pl.when(p > 0)
                def _():
                    pltpu.make_async_copy(
                        row_ref, g_h.at[qbase], osems[buf]).wait()

                pltpu.async_copy(t1_h.at[idx_ref], row_ref, sems[buf])
            return 0

        lax.fori_loop(0, QPW // 2, per_pair, 0)
        for buf in range(2):
            pltpu.make_async_copy(
                t1_h.at[idxb.at[buf]], rows.at[buf], sems[buf]).wait()
            pltpu.sync_copy(rows.at[buf], g_h.at[qbase + QPW - 2 + buf])
        pltpu.sync_copy(cnts, cnt_h.at[pl.ds(qbase, QPW)])

    return body(px, py, pz, qx, qy, qz, t1)


def _mlp1_body(feat_ref, pen_ref, w0_ref, b0_ref, w1_ref, b1_ref, w2_ref,
               b2_ref, pos1_ref, wx2_ref, wr2_ref, x1_ref, t1_ref):
    QB = feat_ref.shape[0] // K
    X = feat_ref[...]
    h = jnp.dot(X, w0_ref[...], preferred_element_type=F32) + b0_ref[0:1, :]
    h = jnp.maximum(h, 0.0)
    h = jnp.dot(h, w1_ref[...], preferred_element_type=F32) + b1_ref[0:1, :]
    h = jnp.maximum(h, 0.0)
    h = jnp.dot(h, w2_ref[...], preferred_element_type=F32) + b2_ref[0:1, :]
    h = h + pen_ref[...]
    x1 = jnp.max(h.reshape(QB, K, 128), axis=1)
    x1_ref[...] = x1
    t1_ref[...] = (
        jnp.dot(x1, wx2_ref[...], preferred_element_type=F32)
        + jnp.dot(pos1_ref[...], wr2_ref[...], preferred_element_type=F32))


def _mlp2_body(g_ref, pen_ref, pos2_ref, wr2_ref, b0_ref, w1_ref, b1_ref,
               w2_ref, b2_ref, x2_ref):
    QB, _, D = g_ref.shape
    U = jnp.dot(pos2_ref[...], wr2_ref[...], preferred_element_type=F32)
    h = (g_ref[...]
         - lax.broadcast_in_dim(U, (QB, K, D), (0, 2))
         + lax.broadcast_in_dim(b0_ref[0:1, :], (QB, K, D), (1, 2)))
    h = jnp.maximum(h, 0.0).reshape(QB * K, D)
    h = jnp.dot(h, w1_ref[...], preferred_element_type=F32) + b1_ref[0:1, :]
    h = jnp.maximum(h, 0.0)
    h = jnp.dot(h, w2_ref[...], preferred_element_type=F32) + b2_ref[0:1, :]
    h = h + pen_ref[...]
    x2_ref[...] = jnp.max(h.reshape(QB, K, 256), axis=1)


def _stage3_body(x2_ref, pos2_ref, w3a_ref, w3b_ref, b0_ref, w1_ref, b1_ref,
                 w2_ref, b2_ref, wfc_ref, bfc_ref, out_ref):
    x2 = x2_ref[0]
    p = pos2_ref[0]
    h = (jnp.dot(x2, w3a_ref[...], preferred_element_type=F32)
         + jnp.dot(p, w3b_ref[...], preferred_element_type=F32)
         + b0_ref[0:1, :])
    h = jnp.maximum(h, 0.0)
    h = jnp.dot(h, w1_ref[...], preferred_element_type=F32) + b1_ref[0:1, :]
    h = jnp.maximum(h, 0.0)
    h = jnp.dot(h, w2_ref[...], preferred_element_type=F32) + b2_ref[0:1, :]
    g = jnp.max(h, axis=0, keepdims=True)
    out_ref[0] = jnp.dot(g, wfc_ref[...], preferred_element_type=F32) \
        + bfc_ref[0:1, :]


def _tile8(b):
    return jnp.tile(b.reshape(1, -1), (8, 1))


def _penalty(cnt):
    BQ = cnt.shape[0]
    kio = lax.broadcasted_iota(I32, (BQ, K), 1)
    pen = jnp.where(kio < cnt[:, None], 0.0, -jnp.inf).astype(F32)
    return pen.reshape(BQ * K, 1)


def _mlp1(feat, cnt, pos1p, W0p, b0, W1, b1, W2, b2, Wx2, Wr2p):
    BQ = cnt.shape[0]
    QB = 64
    grid = BQ // QB
    feat = feat.reshape(BQ * K, 8)
    full = lambda shp: pl.BlockSpec(shp, lambda i: (0, 0))
    return pl.pallas_call(
        _mlp1_body,
        grid=(grid,),
        in_specs=[
            pl.BlockSpec((QB * K, 8), lambda i: (i, 0)),
            pl.BlockSpec((QB * K, 1), lambda i: (i, 0)),
            full((8, 64)), full((8, 64)),
            full((64, 64)), full((8, 64)),
            full((64, 128)), full((8, 128)),
            pl.BlockSpec((QB, 8), lambda i: (i, 0)),
            full((128, 128)), full((8, 128)),
        ],
        out_specs=[
            pl.BlockSpec((QB, 128), lambda i: (i, 0)),
            pl.BlockSpec((QB, 128), lambda i: (i, 0)),
        ],
        out_shape=[
            jax.ShapeDtypeStruct((BQ, 128), F32),
            jax.ShapeDtypeStruct((BQ, 128), F32),
        ],
    )(feat, _penalty(cnt), W0p, _tile8(b0), W1, _tile8(b1), W2, _tile8(b2),
      pos1p, Wx2, Wr2p)


def _mlp2(G, cnt, pos2p, Wr2p, b0, W1, b1, W2, b2):
    BQ = G.shape[0]
    QB = 32
    grid = BQ // QB
    full = lambda shp: pl.BlockSpec(shp, lambda i: (0, 0))
    return pl.pallas_call(
        _mlp2_body,
        grid=(grid,),
        in_specs=[
            pl.BlockSpec((QB, K, 128), lambda i: (i, 0, 0)),
            pl.BlockSpec((QB * K, 1), lambda i: (i, 0)),
            pl.BlockSpec((QB, 8), lambda i: (i, 0)),
            full((8, 128)), full((8, 128)),
            full((128, 128)), full((8, 128)),
            full((128, 256)), full((8, 256)),
        ],
        out_specs=pl.BlockSpec((QB, 256), lambda i: (i, 0)),
        out_shape=jax.ShapeDtypeStruct((BQ, 256), F32),
    )(G, _penalty(cnt), pos2p, Wr2p, _tile8(b0), W1, _tile8(b1), W2,
      _tile8(b2))


def _stage3(x2, pos2p, W3a, W3bp, b0, W1, b1, W2, b2, Wfc, bfc):
    B = x2.shape[0]
    full = lambda shp: pl.BlockSpec(shp, lambda i: (0, 0))
    out = pl.pallas_call(
        _stage3_body,
        grid=(B,),
        in_specs=[
            pl.BlockSpec((1, 512, 256), lambda i: (i, 0, 0)),
            pl.BlockSpec((1, 512, 8), lambda i: (i, 0, 0)),
            full((256, 256)), full((8, 256)), full((8, 256)),
            full((256, 512)), full((8, 512)),
            full((512, 1024)), full((8, 1024)),
            full((1024, 256)), full((8, 256)),
        ],
        out_specs=pl.BlockSpec((1, 1, 256), lambda i: (i, 0, 0)),
        out_shape=jax.ShapeDtypeStruct((B, 1, 256), F32),
    )(x2, pos2p, W3a, W3bp, _tile8(b0), W1, _tile8(b1), W2, _tile8(b2),
      Wfc, _tile8(bfc))
    return out[:, 0, :]


def kernel(data, W1_0, b1_0, W1_1, b1_1, W1_2, b1_2, W2_0, b2_0, W2_1, b2_1,
           W2_2, b2_2, W3_0, b3_0, W3_1, b3_1, W3_2, b3_2, Wfc, bfc):
    B, N, _ = data.shape
    Q1, Q2 = N // 2, N // 8

    pos_t = jnp.transpose(data, (0, 2, 1))
    p0 = [pos_t[:, c, :] for c in range(3)]
    p1 = _fps(*p0, Q1)
    p2 = _fps(*p1, Q2)

    q1 = [c.reshape(B * Q1) for c in p1]
    q2 = [c.reshape(B * Q2) for c in p2]

    feat1, cnt1 = _sc_search_gather_sa1(*p0, *q1, F32(0.2 * 0.2))

    pos1p = jnp.pad(jnp.stack(p1, axis=-1).reshape(B * Q1, 3),
                    ((0, 0), (0, 5)))
    pos2p = jnp.pad(jnp.stack(p2, axis=-1).reshape(B * Q2, 3),
                    ((0, 0), (0, 5)))
    W0p = jnp.concatenate([W1_0, jnp.zeros((2, 64), F32)], axis=0)
    Wx2 = W2_0[:128]
    Wr2p = jnp.concatenate([W2_0[128:], jnp.zeros((5, 128), F32)], axis=0)

    x1, T1 = _mlp1(feat1, cnt1, pos1p, W0p, b1_0, W1_1, b1_1, W1_2, b1_2,
                   Wx2, Wr2p)

    G, cnt2 = _sc_search_gather_sa2(*p1, *q2, T1, F32(0.4 * 0.4))
    x2 = _mlp2(G, cnt2, pos2p, Wr2p, b2_0, W2_1, b2_1, W2_2, b2_2)

    W3a = W3_0[:256]
    W3bp = jnp.concatenate([W3_0[256:], jnp.zeros((5, 256), F32)], axis=0)
    out = _stage3(x2.reshape(B, Q2, 256), pos2p.reshape(B, Q2, 8),
                  W3a, W3bp, b3_0, W3_1, b3_1, W3_2, b3_2, Wfc, bfc)
    return out


# speedup vs baseline: 1.4006x; 1.3085x over previous
import functools

import jax
import jax.numpy as jnp
from jax import lax
from jax.experimental import pallas as pl
from jax.experimental.pallas import tpu as pltpu
from jax.experimental.pallas import tpu_sc as plsc

F32 = jnp.float32
I32 = jnp.int32
K = 64


def _fps_body(px_ref, py_ref, pz_ref, ox_ref, oy_ref, oz_ref, *, M):
    B, N = px_ref.shape
    px = px_ref[...]
    py = py_ref[...]
    pz = pz_ref[...]
    iota_n = lax.broadcasted_iota(I32, (B, N), 1)
    iota_m = lax.broadcasted_iota(I32, (B, M), 1)

    def body(i, st):
        dists, sx, sy, sz = st
        m = jnp.max(dists, axis=1, keepdims=True)
        nxt = jnp.min(jnp.where(dists == m, iota_n, N), axis=1,
                      keepdims=True)
        selm = iota_n == nxt
        qx = jnp.sum(jnp.where(selm, px, 0.0), axis=1, keepdims=True)
        qy = jnp.sum(jnp.where(selm, py, 0.0), axis=1, keepdims=True)
        qz = jnp.sum(jnp.where(selm, pz, 0.0), axis=1, keepdims=True)
        ddx = px - qx
        ddy = py - qy
        ddz = pz - qz
        d = (ddx * ddx + ddy * ddy) + ddz * ddz
        put = iota_m == i
        return (jnp.minimum(dists, d), jnp.where(put, qx, sx),
                jnp.where(put, qy, sy), jnp.where(put, qz, sz))

    z = jnp.zeros((B, M), F32)
    _, sx, sy, sz = lax.fori_loop(
        0, M, body, (jnp.full((B, N), 3.0e38, F32), z, z, z))
    ox_ref[...] = sx
    oy_ref[...] = sy
    oz_ref[...] = sz


def _fps(px, py, pz, M):
    B = px.shape[0]
    out = jax.ShapeDtypeStruct((B, M), F32)
    return pl.pallas_call(
        functools.partial(_fps_body, M=M),
        out_shape=[out, out, out],
    )(px, py, pz)


def _iota16():
    return lax.broadcasted_iota(I32, (16,), 0)


def _splat(val, dtype=I32):
    return jnp.full((16,), val, dtype)


def _sc_search_gather_sa1(px, py, pz, qx, qy, qz, r2):
    B, N = px.shape
    BQ = qx.shape[0]
    Q = BQ // B
    NW = 32
    QPW = BQ // NW
    WPB = NW // B
    mesh = plsc.VectorSubcoreMesh(core_axis_name="c", subcore_axis_name="s")

    @functools.partial(
        pl.kernel,
        mesh=mesh,
        compiler_params=pltpu.CompilerParams(needs_layout_passes=False),
        out_type=[
            jax.ShapeDtypeStruct((BQ * K * 8,), F32),
            jax.ShapeDtypeStruct((BQ,), I32),
        ],
        scratch_types=[
            pltpu.VMEM((N,), F32),
            pltpu.VMEM((N,), F32),
            pltpu.VMEM((N,), F32),
            pltpu.VMEM((QPW,), F32),
            pltpu.VMEM((QPW,), F32),
            pltpu.VMEM((QPW,), F32),
            pltpu.VMEM((K * 8,), F32),
            pltpu.VMEM((K * 8,), F32),
            pltpu.VMEM((QPW,), I32),
        ],
    )
    def body(px_h, py_h, pz_h, qx_h, qy_h, qz_h, feat_h, cnt_h,
             pxs, pys, pzs, qxs, qys, qzs, ot0, ot1, cnts):
        wid = lax.axis_index("s") * 2 + lax.axis_index("c")
        b = wid // WPB
        qbase = wid * QPW
        pltpu.sync_copy(px_h.at[b], pxs)
        pltpu.sync_copy(py_h.at[b], pys)
        pltpu.sync_copy(pz_h.at[b], pzs)
        pltpu.sync_copy(qx_h.at[pl.ds(qbase, QPW)], qxs)
        pltpu.sync_copy(qy_h.at[pl.ds(qbase, QPW)], qys)
        pltpu.sync_copy(qz_h.at[pl.ds(qbase, QPW)], qzs)

        zf = jnp.zeros((16,), F32)
        for j in range(K * 8 // 16):
            ot0[pl.ds(j * 16, 16)] = zf
            ot1[pl.ds(j * 16, 16)] = zf

        lanes = _iota16()

        def per_pair(p, _):
            q0 = p * 2
            iq0 = _splat(q0)
            iq1 = iq0 + 1
            qxv0 = plsc.load_gather(qxs, [iq0])
            qyv0 = plsc.load_gather(qys, [iq0])
            qzv0 = plsc.load_gather(qzs, [iq0])
            qxv1 = plsc.load_gather(qxs, [iq1])
            qyv1 = plsc.load_gather(qys, [iq1])
            qzv1 = plsc.load_gather(qzs, [iq1])

            def chunk(ci, carry):
                cnt0, cnt1 = carry
                base = ci * 16
                pxv = pxs[pl.ds(base, 16)]
                pyv = pys[pl.ds(base, 16)]
                pzv = pzs[pl.ds(base, 16)]
                dx0 = pxv - qxv0
                dy0 = pyv - qyv0
                dz0 = pzv - qzv0
                dx1 = pxv - qxv1
                dy1 = pyv - qyv1
                dz1 = pzv - qzv1
                d20 = (dx0 * dx0 + dy0 * dy0) + dz0 * dz0
                d21 = (dx1 * dx1 + dy1 * dy1) + dz1 * dz1
                m0 = d20 <= r2
                m1 = d21 <= r2
                s0 = cnt0 + plsc.cumsum(m0.astype(I32)) - 1
                s1 = cnt1 + plsc.cumsum(m1.astype(I32)) - 1
                ok0 = m0 & (s0 < K)
                ok1 = m1 & (s1 < K)
                f0 = s0 * 8
                f1 = s1 * 8
                plsc.store_scatter(ot0, [f0], pxv, mask=ok0)
                plsc.store_scatter(ot1, [f1], pxv, mask=ok1)
                plsc.store_scatter(ot0, [f0 + 1], pyv, mask=ok0)
                plsc.store_scatter(ot1, [f1 + 1], pyv, mask=ok1)
                plsc.store_scatter(ot0, [f0 + 2], pzv, mask=ok0)
                plsc.store_scatter(ot1, [f1 + 2], pzv, mask=ok1)
                plsc.store_scatter(ot0, [f0 + 3], dx0, mask=ok0)
                plsc.store_scatter(ot1, [f1 + 3], dx1, mask=ok1)
                plsc.store_scatter(ot0, [f0 + 4], dy0, mask=ok0)
                plsc.store_scatter(ot1, [f1 + 4], dy1, mask=ok1)
                plsc.store_scatter(ot0, [f0 + 5], dz0, mask=ok0)
                plsc.store_scatter(ot1, [f1 + 5], dz1, mask=ok1)
                return (cnt0 + plsc.all_reduce_population_count(m0),
                        cnt1 + plsc.all_reduce_population_count(m1))

            z16 = jnp.zeros((16,), I32)
            cnt0, cnt1 = lax.fori_loop(0, N // 16, chunk, (z16, z16))
            plsc.store_scatter(cnts, [iq0], jnp.minimum(cnt0, K),
                               mask=lanes == 0)
            plsc.store_scatter(cnts, [iq1], jnp.minimum(cnt1, K),
                               mask=lanes == 0)
            pltpu.sync_copy(ot0, feat_h.at[pl.ds((qbase + q0) * K * 8,
                                                 K * 8)])
            pltpu.sync_copy(ot1, feat_h.at[pl.ds((qbase + q0 + 1) * K * 8,
                                                 K * 8)])
            return 0

        lax.fori_loop(0, QPW // 2, per_pair, 0)
        pltpu.sync_copy(cnts, cnt_h.at[pl.ds(qbase, QPW)])

    return body(px, py, pz, qx, qy, qz)


def _sc_search_gather_sa2(px, py, pz, qx, qy, qz, t1, r2):
    B, N = px.shape
    BQ = qx.shape[0]
    NW = 32
    QPW = BQ // NW
    WPB = NW // B
    D = t1.shape[1]
    mesh = plsc.VectorSubcoreMesh(core_axis_name="c", subcore_axis_name="s")

    @functools.partial(
        pl.kernel,
        mesh=mesh,
        compiler_params=pltpu.CompilerParams(needs_layout_passes=False),
        out_type=[
            jax.ShapeDtypeStruct((BQ, K, D), F32),
            jax.ShapeDtypeStruct((BQ,), I32),
        ],
        scratch_types=[
            pltpu.VMEM((N,), F32),
            pltpu.VMEM((N,), F32),
            pltpu.VMEM((N,), F32),
            pltpu.VMEM((QPW,), F32),
            pltpu.VMEM((QPW,), F32),
            pltpu.VMEM((QPW,), F32),
            pltpu.VMEM((K,), I32),
            pltpu.VMEM((K, D), F32),
            pltpu.VMEM((QPW,), I32),
            pltpu.SemaphoreType.DMA,
        ],
    )
    def body(px_h, py_h, pz_h, qx_h, qy_h, qz_h, t1_h, g_h, cnt_h,
             pxs, pys, pzs, qxs, qys, qzs, idxb, rows, cnts, sem0):
        wid = lax.axis_index("s") * 2 + lax.axis_index("c")
        b = wid // WPB
        qbase = wid * QPW
        pbase = b * N
        pltpu.sync_copy(px_h.at[b], pxs)
        pltpu.sync_copy(py_h.at[b], pys)
        pltpu.sync_copy(pz_h.at[b], pzs)
        pltpu.sync_copy(qx_h.at[pl.ds(qbase, QPW)], qxs)
        pltpu.sync_copy(qy_h.at[pl.ds(qbase, QPW)], qys)
        pltpu.sync_copy(qz_h.at[pl.ds(qbase, QPW)], qzs)

        zi = jnp.zeros((16,), I32)
        for j in range(K // 16):
            idxb[pl.ds(j * 16, 16)] = zi + pbase

        lanes = _iota16()

        pltpu.async_copy(t1_h.at[idxb], rows, sem0).wait()

        def per_query(q, _):
            iq = _splat(q)
            qxv = plsc.load_gather(qxs, [iq])
            qyv = plsc.load_gather(qys, [iq])
            qzv = plsc.load_gather(qzs, [iq])

            def chunk(ci, cnt_v):
                base = ci * 16
                pxv = pxs[pl.ds(base, 16)]
                pyv = pys[pl.ds(base, 16)]
                pzv = pzs[pl.ds(base, 16)]
                dx = pxv - qxv
                dy = pyv - qyv
                dz = pzv - qzv
                d2 = (dx * dx + dy * dy) + dz * dz
                m = d2 <= r2
                slot = cnt_v + plsc.cumsum(m.astype(I32)) - 1
                ok = m & (slot < K)
                plsc.store_scatter(idxb, [slot],
                                   lanes + (base + pbase), mask=ok)
                return cnt_v + plsc.all_reduce_population_count(m)

            cnt_v = lax.fori_loop(0, N // 16, chunk, jnp.zeros((16,), I32))
            cnt_v = jnp.minimum(cnt_v, K)
            plsc.store_scatter(cnts, [iq], cnt_v, mask=lanes == 0)

            cnt_s = jnp.max(cnt_v)

            @pl.when(cnt_s <= 16)
            def _():
                pltpu.async_copy(t1_h.at[idxb.at[pl.ds(0, 16)]],
                                 rows.at[pl.ds(0, 16)], sem0).wait()

            @pl.when((cnt_s > 16) & (cnt_s <= 32))
            def _():
                pltpu.async_copy(t1_h.at[idxb.at[pl.ds(0, 32)]],
                                 rows.at[pl.ds(0, 32)], sem0).wait()

            @pl.when((cnt_s > 32) & (cnt_s <= 48))
            def _():
                pltpu.async_copy(t1_h.at[idxb.at[pl.ds(0, 48)]],
                                 rows.at[pl.ds(0, 48)], sem0).wait()

            @pl.when(cnt_s > 48)
            def _():
                pltpu.async_copy(t1_h.at[idxb], rows, sem0).wait()

            pltpu.sync_copy(rows, g_h.at[qbase + q])
            return 0

        lax.fori_loop(0, QPW, per_query, 0)
        pltpu.sync_copy(cnts, cnt_h.at[pl.ds(qbase, QPW)])

    return body(px, py, pz, qx, qy, qz, t1)


def _mlp1_body(feat_ref, pen_ref, w0_ref, b0_ref, w1_ref, b1_ref, w2_ref,
               b2_ref, pos1_ref, wx2_ref, wr2_ref, x1_ref, t1_ref):
    QB = feat_ref.shape[0] // K
    X = feat_ref[...]
    h = jnp.dot(X, w0_ref[...], preferred_element_type=F32) + b0_ref[0:1, :]
    h = jnp.maximum(h, 0.0)
    h = jnp.dot(h, w1_ref[...], preferred_element_type=F32) + b1_ref[0:1, :]
    h = jnp.maximum(h, 0.0)
    h = jnp.dot(h, w2_ref[...], preferred_element_type=F32) + b2_ref[0:1, :]
    h = h + pen_ref[...]
    x1 = jnp.max(h.reshape(QB, K, 128), axis=1)
    x1_ref[...] = x1
    t1_ref[...] = (
        jnp.dot(x1, wx2_ref[...], preferred_element_type=F32)
        + jnp.dot(pos1_ref[...], wr2_ref[...], preferred_element_type=F32))


def _mlp2_body(g_ref, pen_ref, pos2_ref, wr2_ref, b0_ref, w1_ref, b1_ref,
               w2_ref, b2_ref, x2_ref):
    QB, _, D = g_ref.shape
    U = jnp.dot(pos2_ref[...], wr2_ref[...], preferred_element_type=F32)
    h = (g_ref[...]
         - lax.broadcast_in_dim(U, (QB, K, D), (0, 2))
         + lax.broadcast_in_dim(b0_ref[0:1, :], (QB, K, D), (1, 2)))
    h = jnp.maximum(h, 0.0).reshape(QB * K, D)
    h = jnp.dot(h, w1_ref[...], preferred_element_type=F32) + b1_ref[0:1, :]
    h = jnp.maximum(h, 0.0)
    h = jnp.dot(h, w2_ref[...], preferred_element_type=F32) + b2_ref[0:1, :]
    h = h + pen_ref[...]
    x2_ref[...] = jnp.max(h.reshape(QB, K, 256), axis=1)


def _stage3_body(x2_ref, pos2_ref, w3a_ref, w3b_ref, b0_ref, w1_ref, b1_ref,
                 w2_ref, b2_ref, wfc_ref, bfc_ref, out_ref):
    x2 = x2_ref[0]
    p = pos2_ref[0]
    h = (jnp.dot(x2, w3a_ref[...], preferred_element_type=F32)
         + jnp.dot(p, w3b_ref[...], preferred_element_type=F32)
         + b0_ref[0:1, :])
    h = jnp.maximum(h, 0.0)
    h = jnp.dot(h, w1_ref[...], preferred_element_type=F32) + b1_ref[0:1, :]
    h = jnp.maximum(h, 0.0)
    h = jnp.dot(h, w2_ref[...], preferred_element_type=F32) + b2_ref[0:1, :]
    g = jnp.max(h, axis=0, keepdims=True)
    out_ref[0] = jnp.dot(g, wfc_ref[...], preferred_element_type=F32) \
        + bfc_ref[0:1, :]


def _tile8(b):
    return jnp.tile(b.reshape(1, -1), (8, 1))


def _penalty(cnt):
    BQ = cnt.shape[0]
    kio = lax.broadcasted_iota(I32, (BQ, K), 1)
    pen = jnp.where(kio < cnt[:, None], 0.0, -jnp.inf).astype(F32)
    return pen.reshape(BQ * K, 1)


def _mlp1(feat, cnt, pos1p, W0p, b0, W1, b1, W2, b2, Wx2, Wr2p):
    BQ = cnt.shape[0]
    QB = 64
    grid = BQ // QB
    feat = feat.reshape(BQ * K, 8)
    full = lambda shp: pl.BlockSpec(shp, lambda i: (0, 0))
    return pl.pallas_call(
        _mlp1_body,
        grid=(grid,),
        in_specs=[
            pl.BlockSpec((QB * K, 8), lambda i: (i, 0)),
            pl.BlockSpec((QB * K, 1), lambda i: (i, 0)),
            full((8, 64)), full((8, 64)),
            full((64, 64)), full((8, 64)),
            full((64, 128)), full((8, 128)),
            pl.BlockSpec((QB, 8), lambda i: (i, 0)),
            full((128, 128)), full((8, 128)),
        ],
        out_specs=[
            pl.BlockSpec((QB, 128), lambda i: (i, 0)),
            pl.BlockSpec((QB, 128), lambda i: (i, 0)),
        ],
        out_shape=[
            jax.ShapeDtypeStruct((BQ, 128), F32),
            jax.ShapeDtypeStruct((BQ, 128), F32),
        ],
    )(feat, _penalty(cnt), W0p, _tile8(b0), W1, _tile8(b1), W2, _tile8(b2),
      pos1p, Wx2, Wr2p)


def _mlp2(G, cnt, pos2p, Wr2p, b0, W1, b1, W2, b2):
    BQ = G.shape[0]
    QB = 32
    grid = BQ // QB
    full = lambda shp: pl.BlockSpec(shp, lambda i: (0, 0))
    return pl.pallas_call(
        _mlp2_body,
        grid=(grid,),
        in_specs=[
            pl.BlockSpec((QB, K, 128), lambda i: (i, 0, 0)),
            pl.BlockSpec((QB * K, 1), lambda i: (i, 0)),
            pl.BlockSpec((QB, 8), lambda i: (i, 0)),
            full((8, 128)), full((8, 128)),
            full((128, 128)), full((8, 128)),
            full((128, 256)), full((8, 256)),
        ],
        out_specs=pl.BlockSpec((QB, 256), lambda i: (i, 0)),
        out_shape=jax.ShapeDtypeStruct((BQ, 256), F32),
    )(G, _penalty(cnt), pos2p, Wr2p, _tile8(b0), W1, _tile8(b1), W2,
      _tile8(b2))


def _stage3(x2, pos2p, W3a, W3bp, b0, W1, b1, W2, b2, Wfc, bfc):
    B = x2.shape[0]
    full = lambda shp: pl.BlockSpec(shp, lambda i: (0, 0))
    out = pl.pallas_call(
        _stage3_body,
        grid=(B,),
        in_specs=[
            pl.BlockSpec((1, 512, 256), lambda i: (i, 0, 0)),
            pl.BlockSpec((1, 512, 8), lambda i: (i, 0, 0)),
            full((256, 256)), full((8, 256)), full((8, 256)),
            full((256, 512)), full((8, 512)),
            full((512, 1024)), full((8, 1024)),
            full((1024, 256)), full((8, 256)),
        ],
        out_specs=pl.BlockSpec((1, 1, 256), lambda i: (i, 0, 0)),
        out_shape=jax.ShapeDtypeStruct((B, 1, 256), F32),
    )(x2, pos2p, W3a, W3bp, _tile8(b0), W1, _tile8(b1), W2, _tile8(b2),
      Wfc, _tile8(bfc))
    return out[:, 0, :]


def kernel(data, W1_0, b1_0, W1_1, b1_1, W1_2, b1_2, W2_0, b2_0, W2_1, b2_1,
           W2_2, b2_2, W3_0, b3_0, W3_1, b3_1, W3_2, b3_2, Wfc, bfc):
    B, N, _ = data.shape
    Q1, Q2 = N // 2, N // 8

    pos_t = jnp.transpose(data, (0, 2, 1))
    p0 = [pos_t[:, c, :] for c in range(3)]
    p1 = _fps(*p0, Q1)
    p2 = _fps(*p1, Q2)

    q1 = [c.reshape(B * Q1) for c in p1]
    q2 = [c.reshape(B * Q2) for c in p2]

    feat1, cnt1 = _sc_search_gather_sa1(*p0, *q1, F32(0.2 * 0.2))

    pos1p = jnp.pad(jnp.stack(p1, axis=-1).reshape(B * Q1, 3),
                    ((0, 0), (0, 5)))
    pos2p = jnp.pad(jnp.stack(p2, axis=-1).reshape(B * Q2, 3),
                    ((0, 0), (0, 5)))
    W0p = jnp.concatenate([W1_0, jnp.zeros((2, 64), F32)], axis=0)
    Wx2 = W2_0[:128]
    Wr2p = jnp.concatenate([W2_0[128:], jnp.zeros((5, 128), F32)], axis=0)

    x1, T1 = _mlp1(feat1, cnt1, pos1p, W0p, b1_0, W1_1, b1_1, W1_2, b1_2,
                   Wx2, Wr2p)

    G, cnt2 = _sc_search_gather_sa2(*p1, *q2, T1, F32(0.4 * 0.4))
    x2 = _mlp2(G, cnt2, pos2p, Wr2p, b2_0, W2_1, b2_1, W2_2, b2_2)

    W3a = W3_0[:256]
    W3bp = jnp.concatenate([W3_0[256:], jnp.zeros((5, 256), F32)], axis=0)
    out = _stage3(x2.reshape(B, Q2, 256), pos2p.reshape(B, Q2, 8),
                  W3a, W3bp, b3_0, W3_1, b3_1, W3_2, b3_2, Wfc, bfc)
    return out


# speedup vs baseline: 1.4196x; 1.0136x over previous
import functools

import jax
import jax.numpy as jnp
from jax import lax
from jax.experimental import pallas as pl
from jax.experimental.pallas import tpu as pltpu
from jax.experimental.pallas import tpu_sc as plsc

F32 = jnp.float32
I32 = jnp.int32
K = 64


def _fps_body(px_ref, py_ref, pz_ref, ox_ref, oy_ref, oz_ref, *, M):
    B, N = px_ref.shape
    px = px_ref[...]
    py = py_ref[...]
    pz = pz_ref[...]
    iota_n = lax.broadcasted_iota(I32, (B, N), 1)
    iota_m = lax.broadcasted_iota(I32, (B, M), 1)

    def body(i, st):
        dists, sx, sy, sz = st
        m = jnp.max(dists, axis=1, keepdims=True)
        nxt = jnp.min(jnp.where(dists == m, iota_n, N), axis=1,
                      keepdims=True)
        selm = iota_n == nxt
        qx = jnp.sum(jnp.where(selm, px, 0.0), axis=1, keepdims=True)
        qy = jnp.sum(jnp.where(selm, py, 0.0), axis=1, keepdims=True)
        qz = jnp.sum(jnp.where(selm, pz, 0.0), axis=1, keepdims=True)
        ddx = px - qx
        ddy = py - qy
        ddz = pz - qz
        d = (ddx * ddx + ddy * ddy) + ddz * ddz
        put = iota_m == i
        return (jnp.minimum(dists, d), jnp.where(put, qx, sx),
                jnp.where(put, qy, sy), jnp.where(put, qz, sz))

    z = jnp.zeros((B, M), F32)
    _, sx, sy, sz = lax.fori_loop(
        0, M, body, (jnp.full((B, N), 3.0e38, F32), z, z, z))
    ox_ref[...] = sx
    oy_ref[...] = sy
    oz_ref[...] = sz


def _fps(px, py, pz, M):
    B = px.shape[0]
    out = jax.ShapeDtypeStruct((B, M), F32)
    return pl.pallas_call(
        functools.partial(_fps_body, M=M),
        out_shape=[out, out, out],
    )(px, py, pz)


def _iota16():
    return lax.broadcasted_iota(I32, (16,), 0)


def _splat(val, dtype=I32):
    return jnp.full((16,), val, dtype)


def _sc_search_gather_sa1(px, py, pz, qx, qy, qz, r2):
    B, N = px.shape
    BQ = qx.shape[0]
    Q = BQ // B
    NW = 32
    QPW = BQ // NW
    WPB = NW // B
    mesh = plsc.VectorSubcoreMesh(core_axis_name="c", subcore_axis_name="s")

    @functools.partial(
        pl.kernel,
        mesh=mesh,
        compiler_params=pltpu.CompilerParams(needs_layout_passes=False),
        out_type=[
            jax.ShapeDtypeStruct((BQ * K * 8,), F32),
            jax.ShapeDtypeStruct((BQ,), I32),
        ],
        scratch_types=[
            pltpu.VMEM((N,), F32),
            pltpu.VMEM((N,), F32),
            pltpu.VMEM((N,), F32),
            pltpu.VMEM((QPW,), F32),
            pltpu.VMEM((QPW,), F32),
            pltpu.VMEM((QPW,), F32),
            pltpu.VMEM((K * 8,), F32),
            pltpu.VMEM((K * 8,), F32),
            pltpu.VMEM((QPW,), I32),
        ],
    )
    def body(px_h, py_h, pz_h, qx_h, qy_h, qz_h, feat_h, cnt_h,
             pxs, pys, pzs, qxs, qys, qzs, ot0, ot1, cnts):
        wid = lax.axis_index("s") * 2 + lax.axis_index("c")
        b = wid // WPB
        qbase = wid * QPW
        pltpu.sync_copy(px_h.at[b], pxs)
        pltpu.sync_copy(py_h.at[b], pys)
        pltpu.sync_copy(pz_h.at[b], pzs)
        pltpu.sync_copy(qx_h.at[pl.ds(qbase, QPW)], qxs)
        pltpu.sync_copy(qy_h.at[pl.ds(qbase, QPW)], qys)
        pltpu.sync_copy(qz_h.at[pl.ds(qbase, QPW)], qzs)

        zf = jnp.zeros((16,), F32)
        for j in range(K * 8 // 16):
            ot0[pl.ds(j * 16, 16)] = zf
            ot1[pl.ds(j * 16, 16)] = zf

        lanes = _iota16()

        def per_pair(p, _):
            q0 = p * 2
            iq0 = _splat(q0)
            iq1 = iq0 + 1
            qxv0 = plsc.load_gather(qxs, [iq0])
            qyv0 = plsc.load_gather(qys, [iq0])
            qzv0 = plsc.load_gather(qzs, [iq0])
            qxv1 = plsc.load_gather(qxs, [iq1])
            qyv1 = plsc.load_gather(qys, [iq1])
            qzv1 = plsc.load_gather(qzs, [iq1])

            def chunk(ci, carry):
                cnt0, cnt1 = carry
                base = ci * 16
                pxv = pxs[pl.ds(base, 16)]
                pyv = pys[pl.ds(base, 16)]
                pzv = pzs[pl.ds(base, 16)]
                dx0 = pxv - qxv0
                dy0 = pyv - qyv0
                dz0 = pzv - qzv0
                dx1 = pxv - qxv1
                dy1 = pyv - qyv1
                dz1 = pzv - qzv1
                d20 = (dx0 * dx0 + dy0 * dy0) + dz0 * dz0
                d21 = (dx1 * dx1 + dy1 * dy1) + dz1 * dz1
                m0 = d20 <= r2
                m1 = d21 <= r2
                s0 = cnt0 + plsc.cumsum(m0.astype(I32)) - 1
                s1 = cnt1 + plsc.cumsum(m1.astype(I32)) - 1
                ok0 = m0 & (s0 < K)
                ok1 = m1 & (s1 < K)
                f0 = s0 * 8
                f1 = s1 * 8
                plsc.store_scatter(ot0, [f0], pxv, mask=ok0)
                plsc.store_scatter(ot1, [f1], pxv, mask=ok1)
                plsc.store_scatter(ot0, [f0 + 1], pyv, mask=ok0)
                plsc.store_scatter(ot1, [f1 + 1], pyv, mask=ok1)
                plsc.store_scatter(ot0, [f0 + 2], pzv, mask=ok0)
                plsc.store_scatter(ot1, [f1 + 2], pzv, mask=ok1)
                plsc.store_scatter(ot0, [f0 + 3], dx0, mask=ok0)
                plsc.store_scatter(ot1, [f1 + 3], dx1, mask=ok1)
                plsc.store_scatter(ot0, [f0 + 4], dy0, mask=ok0)
                plsc.store_scatter(ot1, [f1 + 4], dy1, mask=ok1)
                plsc.store_scatter(ot0, [f0 + 5], dz0, mask=ok0)
                plsc.store_scatter(ot1, [f1 + 5], dz1, mask=ok1)
                return (cnt0 + plsc.all_reduce_population_count(m0),
                        cnt1 + plsc.all_reduce_population_count(m1))

            z16 = jnp.zeros((16,), I32)
            cnt0, cnt1 = lax.fori_loop(0, N // 16, chunk, (z16, z16))
            cnt0 = jnp.minimum(cnt0, K)
            cnt1 = jnp.minimum(cnt1, K)
            plsc.store_scatter(cnts, [iq0], cnt0, mask=lanes == 0)
            plsc.store_scatter(cnts, [iq1], cnt1, mask=lanes == 0)

            def copy_out(ot, cnt_v, row):
                cs = jnp.max(cnt_v)
                base = row * (K * 8)

                @pl.when(cs <= 16)
                def _():
                    pltpu.sync_copy(ot.at[pl.ds(0, 128)],
                                    feat_h.at[pl.ds(base, 128)])

                @pl.when((cs > 16) & (cs <= 32))
                def _():
                    pltpu.sync_copy(ot.at[pl.ds(0, 256)],
                                    feat_h.at[pl.ds(base, 256)])

                @pl.when((cs > 32) & (cs <= 48))
                def _():
                    pltpu.sync_copy(ot.at[pl.ds(0, 384)],
                                    feat_h.at[pl.ds(base, 384)])

                @pl.when(cs > 48)
                def _():
                    pltpu.sync_copy(ot, feat_h.at[pl.ds(base, K * 8)])

            copy_out(ot0, cnt0, qbase + q0)
            copy_out(ot1, cnt1, qbase + q0 + 1)
            return 0

        lax.fori_loop(0, QPW // 2, per_pair, 0)
        pltpu.sync_copy(cnts, cnt_h.at[pl.ds(qbase, QPW)])

    return body(px, py, pz, qx, qy, qz)


def _sc_search_gather_sa2(px, py, pz, qx, qy, qz, t1, r2):
    B, N = px.shape
    BQ = qx.shape[0]
    NW = 32
    QPW = BQ // NW
    WPB = NW // B
    D = t1.shape[1]
    mesh = plsc.VectorSubcoreMesh(core_axis_name="c", subcore_axis_name="s")

    @functools.partial(
        pl.kernel,
        mesh=mesh,
        compiler_params=pltpu.CompilerParams(needs_layout_passes=False),
        out_type=[
            jax.ShapeDtypeStruct((BQ, K, D), F32),
            jax.ShapeDtypeStruct((BQ,), I32),
        ],
        scratch_types=[
            pltpu.VMEM((N,), F32),
            pltpu.VMEM((N,), F32),
            pltpu.VMEM((N,), F32),
            pltpu.VMEM((QPW,), F32),
            pltpu.VMEM((QPW,), F32),
            pltpu.VMEM((QPW,), F32),
            pltpu.VMEM((K,), I32),
            pltpu.VMEM((K, D), F32),
            pltpu.VMEM((QPW,), I32),
            pltpu.SemaphoreType.DMA,
        ],
    )
    def body(px_h, py_h, pz_h, qx_h, qy_h, qz_h, t1_h, g_h, cnt_h,
             pxs, pys, pzs, qxs, qys, qzs, idxb, rows, cnts, sem0):
        wid = lax.axis_index("s") * 2 + lax.axis_index("c")
        b = wid // WPB
        qbase = wid * QPW
        pbase = b * N
        pltpu.sync_copy(px_h.at[b], pxs)
        pltpu.sync_copy(py_h.at[b], pys)
        pltpu.sync_copy(pz_h.at[b], pzs)
        pltpu.sync_copy(qx_h.at[pl.ds(qbase, QPW)], qxs)
        pltpu.sync_copy(qy_h.at[pl.ds(qbase, QPW)], qys)
        pltpu.sync_copy(qz_h.at[pl.ds(qbase, QPW)], qzs)

        zi = jnp.zeros((16,), I32)
        for j in range(K // 16):
            idxb[pl.ds(j * 16, 16)] = zi + pbase

        lanes = _iota16()

        pltpu.async_copy(t1_h.at[idxb], rows, sem0).wait()

        def per_query(q, _):
            iq = _splat(q)
            qxv = plsc.load_gather(qxs, [iq])
            qyv = plsc.load_gather(qys, [iq])
            qzv = plsc.load_gather(qzs, [iq])

            def chunk(ci, cnt_v):
                base = ci * 16
                pxv = pxs[pl.ds(base, 16)]
                pyv = pys[pl.ds(base, 16)]
                pzv = pzs[pl.ds(base, 16)]
                dx = pxv - qxv
                dy = pyv - qyv
                dz = pzv - qzv
                d2 = (dx * dx + dy * dy) + dz * dz
                m = d2 <= r2
                slot = cnt_v + plsc.cumsum(m.astype(I32)) - 1
                ok = m & (slot < K)
                plsc.store_scatter(idxb, [slot],
                                   lanes + (base + pbase), mask=ok)
                return cnt_v + plsc.all_reduce_population_count(m)

            cnt_v = lax.fori_loop(0, N // 16, chunk, jnp.zeros((16,), I32))
            cnt_v = jnp.minimum(cnt_v, K)
            plsc.store_scatter(cnts, [iq], cnt_v, mask=lanes == 0)

            cnt_s = jnp.max(cnt_v)

            @pl.when(cnt_s <= 16)
            def _():
                pltpu.async_copy(t1_h.at[idxb.at[pl.ds(0, 16)]],
                                 rows.at[pl.ds(0, 16)], sem0).wait()
                pltpu.sync_copy(rows.at[pl.ds(0, 16)],
                                g_h.at[qbase + q, pl.ds(0, 16)])

            @pl.when((cnt_s > 16) & (cnt_s <= 32))
            def _():
                pltpu.async_copy(t1_h.at[idxb.at[pl.ds(0, 32)]],
                                 rows.at[pl.ds(0, 32)], sem0).wait()
                pltpu.sync_copy(rows.at[pl.ds(0, 32)],
                                g_h.at[qbase + q, pl.ds(0, 32)])

            @pl.when((cnt_s > 32) & (cnt_s <= 48))
            def _():
                pltpu.async_copy(t1_h.at[idxb.at[pl.ds(0, 48)]],
                                 rows.at[pl.ds(0, 48)], sem0).wait()
                pltpu.sync_copy(rows.at[pl.ds(0, 48)],
                                g_h.at[qbase + q, pl.ds(0, 48)])

            @pl.when(cnt_s > 48)
            def _():
                pltpu.async_copy(t1_h.at[idxb], rows, sem0).wait()
                pltpu.sync_copy(rows, g_h.at[qbase + q])

            return 0

        lax.fori_loop(0, QPW, per_query, 0)
        pltpu.sync_copy(cnts, cnt_h.at[pl.ds(qbase, QPW)])

    return body(px, py, pz, qx, qy, qz, t1)


def _mlp1_body(feat_ref, pen_ref, cnt_ref, w0_ref, b0_ref, w1_ref, b1_ref,
               w2_ref, b2_ref, pos1_ref, wx2_ref, wr2_ref, x1_ref, t1_ref):
    QB = feat_ref.shape[0] // K
    cs = jnp.max(cnt_ref[...])

    def compute(KS):
        X = feat_ref[...].reshape(QB, K, 8)[:, :KS, :].reshape(QB * KS, 8)
        pen = pen_ref[...].reshape(QB, K, 1)[:, :KS, :].reshape(QB * KS, 1)
        h = (jnp.dot(X, w0_ref[...], preferred_element_type=F32)
             + b0_ref[0:1, :])
        h = jnp.maximum(h, 0.0)
        h = (jnp.dot(h, w1_ref[...], preferred_element_type=F32)
             + b1_ref[0:1, :])
        h = jnp.maximum(h, 0.0)
        h = (jnp.dot(h, w2_ref[...], preferred_element_type=F32)
             + b2_ref[0:1, :])
        h = jnp.where(pen < 0.0, F32(-3.0e38), h)
        x1 = jnp.max(h.reshape(QB, KS, 128), axis=1)
        x1_ref[...] = x1
        t1_ref[...] = (
            jnp.dot(x1, wx2_ref[...], preferred_element_type=F32)
            + jnp.dot(pos1_ref[...], wr2_ref[...],
                      preferred_element_type=F32))

    @pl.when(cs <= 16)
    def _():
        compute(16)

    @pl.when((cs > 16) & (cs <= 32))
    def _():
        compute(32)

    @pl.when((cs > 32) & (cs <= 48))
    def _():
        compute(48)

    @pl.when(cs > 48)
    def _():
        compute(64)


def _mlp2_body(g_ref, pen_ref, cnt_ref, pos2_ref, wr2_ref, b0_ref, w1_ref,
               b1_ref, w2_ref, b2_ref, x2_ref):
    QB, _, D = g_ref.shape
    cs = jnp.max(cnt_ref[...])
    U = jnp.dot(pos2_ref[...], wr2_ref[...], preferred_element_type=F32)

    def compute(KS):
        h = (g_ref[:, :KS, :]
             - lax.broadcast_in_dim(U, (QB, KS, D), (0, 2))
             + lax.broadcast_in_dim(b0_ref[0:1, :], (QB, KS, D), (1, 2)))
        h = jnp.maximum(h, 0.0).reshape(QB * KS, D)
        h = (jnp.dot(h, w1_ref[...], preferred_element_type=F32)
             + b1_ref[0:1, :])
        h = jnp.maximum(h, 0.0)
        h = (jnp.dot(h, w2_ref[...], preferred_element_type=F32)
             + b2_ref[0:1, :])
        pen = pen_ref[...].reshape(QB, K, 1)[:, :KS, :].reshape(QB * KS, 1)
        h = jnp.where(pen < 0.0, F32(-3.0e38), h)
        x2_ref[...] = jnp.max(h.reshape(QB, KS, 256), axis=1)

    @pl.when(cs <= 16)
    def _():
        compute(16)

    @pl.when((cs > 16) & (cs <= 32))
    def _():
        compute(32)

    @pl.when((cs > 32) & (cs <= 48))
    def _():
        compute(48)

    @pl.when(cs > 48)
    def _():
        compute(64)


def _stage3_body(x2_ref, pos2_ref, w3a_ref, w3b_ref, b0_ref, w1_ref, b1_ref,
                 w2_ref, b2_ref, wfc_ref, bfc_ref, out_ref):
    x2 = x2_ref[0]
    p = pos2_ref[0]
    h = (jnp.dot(x2, w3a_ref[...], preferred_element_type=F32)
         + jnp.dot(p, w3b_ref[...], preferred_element_type=F32)
         + b0_ref[0:1, :])
    h = jnp.maximum(h, 0.0)
    h = jnp.dot(h, w1_ref[...], preferred_element_type=F32) + b1_ref[0:1, :]
    h = jnp.maximum(h, 0.0)
    h = jnp.dot(h, w2_ref[...], preferred_element_type=F32) + b2_ref[0:1, :]
    g = jnp.max(h, axis=0, keepdims=True)
    out_ref[0] = jnp.dot(g, wfc_ref[...], preferred_element_type=F32) \
        + bfc_ref[0:1, :]


def _tile8(b):
    return jnp.tile(b.reshape(1, -1), (8, 1))


def _penalty(cnt):
    BQ = cnt.shape[0]
    kio = lax.broadcasted_iota(I32, (BQ, K), 1)
    pen = jnp.where(kio < cnt[:, None], 0.0, -jnp.inf).astype(F32)
    return pen.reshape(BQ * K, 1)


def _mlp1(feat, cnt, pos1p, W0p, b0, W1, b1, W2, b2, Wx2, Wr2p):
    BQ = cnt.shape[0]
    QB = 64
    grid = BQ // QB
    feat = feat.reshape(BQ * K, 8)
    full = lambda shp: pl.BlockSpec(shp, lambda i: (0, 0))
    return pl.pallas_call(
        _mlp1_body,
        grid=(grid,),
        in_specs=[
            pl.BlockSpec((QB * K, 8), lambda i: (i, 0)),
            pl.BlockSpec((QB * K, 1), lambda i: (i, 0)),
            pl.BlockSpec((QB, 1), lambda i: (i, 0)),
            full((8, 64)), full((8, 64)),
            full((64, 64)), full((8, 64)),
            full((64, 128)), full((8, 128)),
            pl.BlockSpec((QB, 8), lambda i: (i, 0)),
            full((128, 128)), full((8, 128)),
        ],
        out_specs=[
            pl.BlockSpec((QB, 128), lambda i: (i, 0)),
            pl.BlockSpec((QB, 128), lambda i: (i, 0)),
        ],
        out_shape=[
            jax.ShapeDtypeStruct((BQ, 128), F32),
            jax.ShapeDtypeStruct((BQ, 128), F32),
        ],
    )(feat, _penalty(cnt), cnt.reshape(BQ, 1), W0p, _tile8(b0), W1,
      _tile8(b1), W2, _tile8(b2), pos1p, Wx2, Wr2p)


def _mlp2(G, cnt, pos2p, Wr2p, b0, W1, b1, W2, b2):
    BQ = G.shape[0]
    QB = 32
    grid = BQ // QB
    full = lambda shp: pl.BlockSpec(shp, lambda i: (0, 0))
    return pl.pallas_call(
        _mlp2_body,
        grid=(grid,),
        in_specs=[
            pl.BlockSpec((QB, K, 128), lambda i: (i, 0, 0)),
            pl.BlockSpec((QB * K, 1), lambda i: (i, 0)),
            pl.BlockSpec((QB, 1), lambda i: (i, 0)),
            pl.BlockSpec((QB, 8), lambda i: (i, 0)),
            full((8, 128)), full((8, 128)),
            full((128, 128)), full((8, 128)),
            full((128, 256)), full((8, 256)),
        ],
        out_specs=pl.BlockSpec((QB, 256), lambda i: (i, 0)),
        out_shape=jax.ShapeDtypeStruct((BQ, 256), F32),
    )(G, _penalty(cnt), cnt.reshape(BQ, 1), pos2p, Wr2p, _tile8(b0), W1,
      _tile8(b1), W2, _tile8(b2))


def _stage3(x2, pos2p, W3a, W3bp, b0, W1, b1, W2, b2, Wfc, bfc):
    B = x2.shape[0]
    full = lambda shp: pl.BlockSpec(shp, lambda i: (0, 0))
    out = pl.pallas_call(
        _stage3_body,
        grid=(B,),
        in_specs=[
            pl.BlockSpec((1, 512, 256), lambda i: (i, 0, 0)),
            pl.BlockSpec((1, 512, 8), lambda i: (i, 0, 0)),
            full((256, 256)), full((8, 256)), full((8, 256)),
            full((256, 512)), full((8, 512)),
            full((512, 1024)), full((8, 1024)),
            full((1024, 256)), full((8, 256)),
        ],
        out_specs=pl.BlockSpec((1, 1, 256), lambda i: (i, 0, 0)),
        out_shape=jax.ShapeDtypeStruct((B, 1, 256), F32),
    )(x2, pos2p, W3a, W3bp, _tile8(b0), W1, _tile8(b1), W2, _tile8(b2),
      Wfc, _tile8(bfc))
    return out[:, 0, :]


def kernel(data, W1_0, b1_0, W1_1, b1_1, W1_2, b1_2, W2_0, b2_0, W2_1, b2_1,
           W2_2, b2_2, W3_0, b3_0, W3_1, b3_1, W3_2, b3_2, Wfc, bfc):
    B, N, _ = data.shape
    Q1, Q2 = N // 2, N // 8

    pos_t = jnp.transpose(data, (0, 2, 1))
    p0 = [pos_t[:, c, :] for c in range(3)]
    p1 = _fps(*p0, Q1)
    p2 = _fps(*p1, Q2)

    q1 = [c.reshape(B * Q1) for c in p1]
    q2 = [c.reshape(B * Q2) for c in p2]

    feat1, cnt1 = _sc_search_gather_sa1(*p0, *q1, F32(0.2 * 0.2))

    pos1p = jnp.pad(jnp.stack(p1, axis=-1).reshape(B * Q1, 3),
                    ((0, 0), (0, 5)))
    pos2p = jnp.pad(jnp.stack(p2, axis=-1).reshape(B * Q2, 3),
                    ((0, 0), (0, 5)))
    W0p = jnp.concatenate([W1_0, jnp.zeros((2, 64), F32)], axis=0)
    Wx2 = W2_0[:128]
    Wr2p = jnp.concatenate([W2_0[128:], jnp.zeros((5, 128), F32)], axis=0)

    x1, T1 = _mlp1(feat1, cnt1, pos1p, W0p, b1_0, W1_1, b1_1, W1_2, b1_2,
                   Wx2, Wr2p)

    G, cnt2 = _sc_search_gather_sa2(*p1, *q2, T1, F32(0.4 * 0.4))
    x2 = _mlp2(G, cnt2, pos2p, Wr2p, b2_0, W2_1, b2_1, W2_2, b2_2)

    W3a = W3_0[:256]
    W3bp = jnp.concatenate([W3_0[256:], jnp.zeros((5, 256), F32)], axis=0)
    out = _stage3(x2.reshape(B, Q2, 256), pos2p.reshape(B, Q2, 8),
                  W3a, W3bp, b3_0, W3_1, b3_1, W3_2, b3_2, Wfc, bfc)
    return out


# speedup vs baseline: 1.5879x; 1.1185x over previous
import functools

import jax
import jax.numpy as jnp
from jax import lax
from jax.experimental import pallas as pl
from jax.experimental.pallas import tpu as pltpu
from jax.experimental.pallas import tpu_sc as plsc

F32 = jnp.float32
I32 = jnp.int32
K = 64


def _fps_body(px_ref, py_ref, pz_ref, ox_ref, oy_ref, oz_ref, *, M):
    B, N = px_ref.shape
    px = px_ref[...]
    py = py_ref[...]
    pz = pz_ref[...]
    iota_n = lax.broadcasted_iota(I32, (B, N), 1)
    iota_m = lax.broadcasted_iota(I32, (B, M), 1)

    def body(i, st):
        dists, sx, sy, sz = st
        m = jnp.max(dists, axis=1, keepdims=True)
        nxt = jnp.min(jnp.where(dists == m, iota_n, N), axis=1,
                      keepdims=True)
        selm = iota_n == nxt
        qx = jnp.sum(jnp.where(selm, px, 0.0), axis=1, keepdims=True)
        qy = jnp.sum(jnp.where(selm, py, 0.0), axis=1, keepdims=True)
        qz = jnp.sum(jnp.where(selm, pz, 0.0), axis=1, keepdims=True)
        ddx = px - qx
        ddy = py - qy
        ddz = pz - qz
        d = (ddx * ddx + ddy * ddy) + ddz * ddz
        put = iota_m == i
        return (jnp.minimum(dists, d), jnp.where(put, qx, sx),
                jnp.where(put, qy, sy), jnp.where(put, qz, sz))

    z = jnp.zeros((B, M), F32)
    _, sx, sy, sz = lax.fori_loop(
        0, M, body, (jnp.full((B, N), 3.0e38, F32), z, z, z))
    ox_ref[...] = sx
    oy_ref[...] = sy
    oz_ref[...] = sz


def _fps(px, py, pz, M):
    B = px.shape[0]
    out = jax.ShapeDtypeStruct((B, M), F32)
    return pl.pallas_call(
        functools.partial(_fps_body, M=M),
        out_shape=[out, out, out],
    )(px, py, pz)


def _iota16():
    return lax.broadcasted_iota(I32, (16,), 0)


def _splat(val, dtype=I32):
    return jnp.full((16,), val, dtype)


def _sc_search_gather_sa1(px, py, pz, qx, qy, qz, r2):
    B, N = px.shape
    BQ = qx.shape[0]
    Q = BQ // B
    NW = 32
    QPW = BQ // NW
    WPB = NW // B
    mesh = plsc.VectorSubcoreMesh(core_axis_name="c", subcore_axis_name="s")

    @functools.partial(
        pl.kernel,
        mesh=mesh,
        compiler_params=pltpu.CompilerParams(needs_layout_passes=False),
        out_type=[
            jax.ShapeDtypeStruct((BQ * K * 8,), F32),
            jax.ShapeDtypeStruct((BQ,), I32),
        ],
        scratch_types=[
            pltpu.VMEM((N,), F32),
            pltpu.VMEM((N,), F32),
            pltpu.VMEM((N,), F32),
            pltpu.VMEM((QPW,), F32),
            pltpu.VMEM((QPW,), F32),
            pltpu.VMEM((QPW,), F32),
            pltpu.VMEM((K * 8,), F32),
            pltpu.VMEM((K * 8,), F32),
            pltpu.VMEM((K * 8,), F32),
            pltpu.VMEM((K * 8,), F32),
            pltpu.VMEM((QPW,), I32),
        ],
    )
    def body(px_h, py_h, pz_h, qx_h, qy_h, qz_h, feat_h, cnt_h,
             pxs, pys, pzs, qxs, qys, qzs, ot0, ot1, ot2, ot3, cnts):
        wid = lax.axis_index("s") * 2 + lax.axis_index("c")
        b = wid // WPB
        qbase = wid * QPW
        pltpu.sync_copy(px_h.at[b], pxs)
        pltpu.sync_copy(py_h.at[b], pys)
        pltpu.sync_copy(pz_h.at[b], pzs)
        pltpu.sync_copy(qx_h.at[pl.ds(qbase, QPW)], qxs)
        pltpu.sync_copy(qy_h.at[pl.ds(qbase, QPW)], qys)
        pltpu.sync_copy(qz_h.at[pl.ds(qbase, QPW)], qzs)

        zf = jnp.zeros((16,), F32)
        ots = [ot0, ot1, ot2, ot3]
        for ot in ots:
            for j in range(K * 8 // 16):
                ot[pl.ds(j * 16, 16)] = zf

        lanes = _iota16()

        def per_quad(p, _):
            q0 = p * 4
            iqs = [_splat(q0 + t) for t in range(4)]
            qxv = [plsc.load_gather(qxs, [iq]) for iq in iqs]
            qyv = [plsc.load_gather(qys, [iq]) for iq in iqs]
            qzv = [plsc.load_gather(qzs, [iq]) for iq in iqs]

            def chunk(ci, carry):
                base = ci * 16
                pxv = pxs[pl.ds(base, 16)]
                pyv = pys[pl.ds(base, 16)]
                pzv = pzs[pl.ds(base, 16)]
                ms = []
                for t in range(4):
                    dx = pxv - qxv[t]
                    dy = pyv - qyv[t]
                    dz = pzv - qzv[t]
                    d2 = (dx * dx + dy * dy) + dz * dz
                    m = d2 <= r2
                    s = carry[t] + plsc.cumsum(m.astype(I32)) - 1
                    ok = m & (s < K)
                    f = s * 8
                    plsc.store_scatter(ots[t], [f], pxv, mask=ok)
                    plsc.store_scatter(ots[t], [f + 1], pyv, mask=ok)
                    plsc.store_scatter(ots[t], [f + 2], pzv, mask=ok)
                    plsc.store_scatter(ots[t], [f + 3], dx, mask=ok)
                    plsc.store_scatter(ots[t], [f + 4], dy, mask=ok)
                    plsc.store_scatter(ots[t], [f + 5], dz, mask=ok)
                    ms.append(m)
                return tuple(
                    carry[t] + plsc.all_reduce_population_count(ms[t])
                    for t in range(4))

            z16 = jnp.zeros((16,), I32)
            cnt4 = lax.fori_loop(0, N // 16, chunk, (z16,) * 4)

            def copy_out(ot, cnt_v, row):
                cs = jnp.max(cnt_v)
                base = row * (K * 8)

                @pl.when(cs <= 16)
                def _():
                    pltpu.sync_copy(ot.at[pl.ds(0, 128)],
                                    feat_h.at[pl.ds(base, 128)])

                @pl.when((cs > 16) & (cs <= 32))
                def _():
                    pltpu.sync_copy(ot.at[pl.ds(0, 256)],
                                    feat_h.at[pl.ds(base, 256)])

                @pl.when((cs > 32) & (cs <= 48))
                def _():
                    pltpu.sync_copy(ot.at[pl.ds(0, 384)],
                                    feat_h.at[pl.ds(base, 384)])

                @pl.when(cs > 48)
                def _():
                    pltpu.sync_copy(ot, feat_h.at[pl.ds(base, K * 8)])

            for t in range(4):
                cv = jnp.minimum(cnt4[t], K)
                plsc.store_scatter(cnts, [iqs[t]], cv, mask=lanes == 0)
                copy_out(ots[t], cv, qbase + q0 + t)
            return 0

        lax.fori_loop(0, QPW // 4, per_quad, 0)
        pltpu.sync_copy(cnts, cnt_h.at[pl.ds(qbase, QPW)])

    return body(px, py, pz, qx, qy, qz)


def _sc_search_gather_sa2(px, py, pz, qx, qy, qz, t1, r2):
    B, N = px.shape
    BQ = qx.shape[0]
    NW = 32
    QPW = BQ // NW
    WPB = NW // B
    D = t1.shape[1]
    mesh = plsc.VectorSubcoreMesh(core_axis_name="c", subcore_axis_name="s")

    @functools.partial(
        pl.kernel,
        mesh=mesh,
        compiler_params=pltpu.CompilerParams(needs_layout_passes=False),
        out_type=[
            jax.ShapeDtypeStruct((BQ, K, D), F32),
            jax.ShapeDtypeStruct((BQ,), I32),
        ],
        scratch_types=[
            pltpu.VMEM((N,), F32),
            pltpu.VMEM((N,), F32),
            pltpu.VMEM((N,), F32),
            pltpu.VMEM((QPW,), F32),
            pltpu.VMEM((QPW,), F32),
            pltpu.VMEM((QPW,), F32),
            pltpu.VMEM((K,), I32),
            pltpu.VMEM((K, D), F32),
            pltpu.VMEM((QPW,), I32),
            pltpu.SemaphoreType.DMA,
        ],
    )
    def body(px_h, py_h, pz_h, qx_h, qy_h, qz_h, t1_h, g_h, cnt_h,
             pxs, pys, pzs, qxs, qys, qzs, idxb, rows, cnts, sem0):
        wid = lax.axis_index("s") * 2 + lax.axis_index("c")
        b = wid // WPB
        qbase = wid * QPW
        pbase = b * N
        pltpu.sync_copy(px_h.at[b], pxs)
        pltpu.sync_copy(py_h.at[b], pys)
        pltpu.sync_copy(pz_h.at[b], pzs)
        pltpu.sync_copy(qx_h.at[pl.ds(qbase, QPW)], qxs)
        pltpu.sync_copy(qy_h.at[pl.ds(qbase, QPW)], qys)
        pltpu.sync_copy(qz_h.at[pl.ds(qbase, QPW)], qzs)

        zi = jnp.zeros((16,), I32)
        for j in range(K // 16):
            idxb[pl.ds(j * 16, 16)] = zi + pbase

        lanes = _iota16()

        pltpu.async_copy(t1_h.at[idxb], rows, sem0).wait()

        def per_query(q, _):
            iq = _splat(q)
            qxv = plsc.load_gather(qxs, [iq])
            qyv = plsc.load_gather(qys, [iq])
            qzv = plsc.load_gather(qzs, [iq])

            def chunk(ci, cnt_v):
                base = ci * 16
                pxv = pxs[pl.ds(base, 16)]
                pyv = pys[pl.ds(base, 16)]
                pzv = pzs[pl.ds(base, 16)]
                dx = pxv - qxv
                dy = pyv - qyv
                dz = pzv - qzv
                d2 = (dx * dx + dy * dy) + dz * dz
                m = d2 <= r2
                slot = cnt_v + plsc.cumsum(m.astype(I32)) - 1
                ok = m & (slot < K)
                plsc.store_scatter(idxb, [slot],
                                   lanes + (base + pbase), mask=ok)
                return cnt_v + plsc.all_reduce_population_count(m)

            cnt_v = lax.fori_loop(0, N // 16, chunk, jnp.zeros((16,), I32))
            cnt_v = jnp.minimum(cnt_v, K)
            plsc.store_scatter(cnts, [iq], cnt_v, mask=lanes == 0)

            cnt_s = jnp.max(cnt_v)

            @pl.when(cnt_s <= 16)
            def _():
                pltpu.async_copy(t1_h.at[idxb.at[pl.ds(0, 16)]],
                                 rows.at[pl.ds(0, 16)], sem0).wait()
                pltpu.sync_copy(rows.at[pl.ds(0, 16)],
                                g_h.at[qbase + q, pl.ds(0, 16)])

            @pl.when((cnt_s > 16) & (cnt_s <= 32))
            def _():
                pltpu.async_copy(t1_h.at[idxb.at[pl.ds(0, 32)]],
                                 rows.at[pl.ds(0, 32)], sem0).wait()
                pltpu.sync_copy(rows.at[pl.ds(0, 32)],
                                g_h.at[qbase + q, pl.ds(0, 32)])

            @pl.when((cnt_s > 32) & (cnt_s <= 48))
            def _():
                pltpu.async_copy(t1_h.at[idxb.at[pl.ds(0, 48)]],
                                 rows.at[pl.ds(0, 48)], sem0).wait()
                pltpu.sync_copy(rows.at[pl.ds(0, 48)],
                                g_h.at[qbase + q, pl.ds(0, 48)])

            @pl.when(cnt_s > 48)
            def _():
                pltpu.async_copy(t1_h.at[idxb], rows, sem0).wait()
                pltpu.sync_copy(rows, g_h.at[qbase + q])

            return 0

        lax.fori_loop(0, QPW, per_query, 0)
        pltpu.sync_copy(cnts, cnt_h.at[pl.ds(qbase, QPW)])

    return body(px, py, pz, qx, qy, qz, t1)


def _mlp1_body(feat_ref, pen_ref, cnt_ref, w0_ref, b0_ref, w1_ref, b1_ref,
               w2_ref, b2_ref, pos1_ref, wx2_ref, wr2_ref, x1_ref, t1_ref):
    QB = feat_ref.shape[0] // K
    cs = jnp.max(cnt_ref[...])

    def compute(KS):
        X = feat_ref[...].reshape(QB, K, 8)[:, :KS, :].reshape(QB * KS, 8)
        pen = pen_ref[...].reshape(QB, K, 1)[:, :KS, :].reshape(QB * KS, 1)
        h = (jnp.dot(X, w0_ref[...], preferred_element_type=F32)
             + b0_ref[0:1, :])
        h = jnp.maximum(h, 0.0)
        h = (jnp.dot(h, w1_ref[...], preferred_element_type=F32)
             + b1_ref[0:1, :])
        h = jnp.maximum(h, 0.0)
        h = (jnp.dot(h, w2_ref[...], preferred_element_type=F32)
             + b2_ref[0:1, :])
        h = jnp.where(pen < 0.0, F32(-3.0e38), h)
        x1 = jnp.max(h.reshape(QB, KS, 128), axis=1)
        x1_ref[...] = x1
        t1_ref[...] = (
            jnp.dot(x1, wx2_ref[...], preferred_element_type=F32)
            + jnp.dot(pos1_ref[...], wr2_ref[...],
                      preferred_element_type=F32))

    @pl.when(cs <= 16)
    def _():
        compute(16)

    @pl.when((cs > 16) & (cs <= 32))
    def _():
        compute(32)

    @pl.when((cs > 32) & (cs <= 48))
    def _():
        compute(48)

    @pl.when(cs > 48)
    def _():
        compute(64)


def _mlp2_body(g_ref, pen_ref, cnt_ref, pos2_ref, wr2_ref, b0_ref, w1_ref,
               b1_ref, w2_ref, b2_ref, x2_ref):
    QB, _, D = g_ref.shape
    cs = jnp.max(cnt_ref[...])
    U = jnp.dot(pos2_ref[...], wr2_ref[...], preferred_element_type=F32)

    def compute(KS):
        h = (g_ref[:, :KS, :]
             - lax.broadcast_in_dim(U, (QB, KS, D), (0, 2))
             + lax.broadcast_in_dim(b0_ref[0:1, :], (QB, KS, D), (1, 2)))
        h = jnp.maximum(h, 0.0).reshape(QB * KS, D)
        h = (jnp.dot(h, w1_ref[...], preferred_element_type=F32)
             + b1_ref[0:1, :])
        h = jnp.maximum(h, 0.0)
        h = (jnp.dot(h, w2_ref[...], preferred_element_type=F32)
             + b2_ref[0:1, :])
        pen = pen_ref[...].reshape(QB, K, 1)[:, :KS, :].reshape(QB * KS, 1)
        h = jnp.where(pen < 0.0, F32(-3.0e38), h)
        x2_ref[...] = jnp.max(h.reshape(QB, KS, 256), axis=1)

    @pl.when(cs <= 16)
    def _():
        compute(16)

    @pl.when((cs > 16) & (cs <= 32))
    def _():
        compute(32)

    @pl.when((cs > 32) & (cs <= 48))
    def _():
        compute(48)

    @pl.when(cs > 48)
    def _():
        compute(64)


def _stage3_body(x2_ref, pos2_ref, w3a_ref, w3b_ref, b0_ref, w1_ref, b1_ref,
                 w2_ref, b2_ref, wfc_ref, bfc_ref, out_ref):
    x2 = x2_ref[0]
    p = pos2_ref[0]
    h = (jnp.dot(x2, w3a_ref[...], preferred_element_type=F32)
         + jnp.dot(p, w3b_ref[...], preferred_element_type=F32)
         + b0_ref[0:1, :])
    h = jnp.maximum(h, 0.0)
    h = jnp.dot(h, w1_ref[...], preferred_element_type=F32) + b1_ref[0:1, :]
    h = jnp.maximum(h, 0.0)
    h = jnp.dot(h, w2_ref[...], preferred_element_type=F32) + b2_ref[0:1, :]
    g = jnp.max(h, axis=0, keepdims=True)
    out_ref[0] = jnp.dot(g, wfc_ref[...], preferred_element_type=F32) \
        + bfc_ref[0:1, :]


def _tile8(b):
    return jnp.tile(b.reshape(1, -1), (8, 1))


def _penalty(cnt):
    BQ = cnt.shape[0]
    kio = lax.broadcasted_iota(I32, (BQ, K), 1)
    pen = jnp.where(kio < cnt[:, None], 0.0, -jnp.inf).astype(F32)
    return pen.reshape(BQ * K, 1)


def _mlp1(feat, cnt, pos1p, W0p, b0, W1, b1, W2, b2, Wx2, Wr2p):
    BQ = cnt.shape[0]
    QB = 64
    grid = BQ // QB
    feat = feat.reshape(BQ * K, 8)
    full = lambda shp: pl.BlockSpec(shp, lambda i: (0, 0))
    return pl.pallas_call(
        _mlp1_body,
        grid=(grid,),
        in_specs=[
            pl.BlockSpec((QB * K, 8), lambda i: (i, 0)),
            pl.BlockSpec((QB * K, 1), lambda i: (i, 0)),
            pl.BlockSpec((QB, 1), lambda i: (i, 0)),
            full((8, 64)), full((8, 64)),
            full((64, 64)), full((8, 64)),
            full((64, 128)), full((8, 128)),
            pl.BlockSpec((QB, 8), lambda i: (i, 0)),
            full((128, 128)), full((8, 128)),
        ],
        out_specs=[
            pl.BlockSpec((QB, 128), lambda i: (i, 0)),
            pl.BlockSpec((QB, 128), lambda i: (i, 0)),
        ],
        out_shape=[
            jax.ShapeDtypeStruct((BQ, 128), F32),
            jax.ShapeDtypeStruct((BQ, 128), F32),
        ],
    )(feat, _penalty(cnt), cnt.reshape(BQ, 1), W0p, _tile8(b0), W1,
      _tile8(b1), W2, _tile8(b2), pos1p, Wx2, Wr2p)


def _mlp2(G, cnt, pos2p, Wr2p, b0, W1, b1, W2, b2):
    BQ = G.shape[0]
    QB = 32
    grid = BQ // QB
    full = lambda shp: pl.BlockSpec(shp, lambda i: (0, 0))
    return pl.pallas_call(
        _mlp2_body,
        grid=(grid,),
        in_specs=[
            pl.BlockSpec((QB, K, 128), lambda i: (i, 0, 0)),
            pl.BlockSpec((QB * K, 1), lambda i: (i, 0)),
            pl.BlockSpec((QB, 1), lambda i: (i, 0)),
            pl.BlockSpec((QB, 8), lambda i: (i, 0)),
            full((8, 128)), full((8, 128)),
            full((128, 128)), full((8, 128)),
            full((128, 256)), full((8, 256)),
        ],
        out_specs=pl.BlockSpec((QB, 256), lambda i: (i, 0)),
        out_shape=jax.ShapeDtypeStruct((BQ, 256), F32),
    )(G, _penalty(cnt), cnt.reshape(BQ, 1), pos2p, Wr2p, _tile8(b0), W1,
      _tile8(b1), W2, _tile8(b2))


def _stage3(x2, pos2p, W3a, W3bp, b0, W1, b1, W2, b2, Wfc, bfc):
    B = x2.shape[0]
    full = lambda shp: pl.BlockSpec(shp, lambda i: (0, 0))
    out = pl.pallas_call(
        _stage3_body,
        grid=(B,),
        in_specs=[
            pl.BlockSpec((1, 512, 256), lambda i: (i, 0, 0)),
            pl.BlockSpec((1, 512, 8), lambda i: (i, 0, 0)),
            full((256, 256)), full((8, 256)), full((8, 256)),
            full((256, 512)), full((8, 512)),
            full((512, 1024)), full((8, 1024)),
            full((1024, 256)), full((8, 256)),
        ],
        out_specs=pl.BlockSpec((1, 1, 256), lambda i: (i, 0, 0)),
        out_shape=jax.ShapeDtypeStruct((B, 1, 256), F32),
    )(x2, pos2p, W3a, W3bp, _tile8(b0), W1, _tile8(b1), W2, _tile8(b2),
      Wfc, _tile8(bfc))
    return out[:, 0, :]


def kernel(data, W1_0, b1_0, W1_1, b1_1, W1_2, b1_2, W2_0, b2_0, W2_1, b2_1,
           W2_2, b2_2, W3_0, b3_0, W3_1, b3_1, W3_2, b3_2, Wfc, bfc):
    B, N, _ = data.shape
    Q1, Q2 = N // 2, N // 8

    pos_t = jnp.transpose(data, (0, 2, 1))
    p0 = [pos_t[:, c, :] for c in range(3)]
    p1 = _fps(*p0, Q1)
    p2 = _fps(*p1, Q2)

    q1 = [c.reshape(B * Q1) for c in p1]
    q2 = [c.reshape(B * Q2) for c in p2]

    feat1, cnt1 = _sc_search_gather_sa1(*p0, *q1, F32(0.2 * 0.2))

    pos1p = jnp.pad(jnp.stack(p1, axis=-1).reshape(B * Q1, 3),
                    ((0, 0), (0, 5)))
    pos2p = jnp.pad(jnp.stack(p2, axis=-1).reshape(B * Q2, 3),
                    ((0, 0), (0, 5)))
    W0p = jnp.concatenate([W1_0, jnp.zeros((2, 64), F32)], axis=0)
    Wx2 = W2_0[:128]
    Wr2p = jnp.concatenate([W2_0[128:], jnp.zeros((5, 128), F32)], axis=0)

    x1, T1 = _mlp1(feat1, cnt1, pos1p, W0p, b1_0, W1_1, b1_1, W1_2, b1_2,
                   Wx2, Wr2p)

    G, cnt2 = _sc_search_gather_sa2(*p1, *q2, T1, F32(0.4 * 0.4))
    x2 = _mlp2(G, cnt2, pos2p, Wr2p, b2_0, W2_1, b2_1, W2_2, b2_2)

    W3a = W3_0[:256]
    W3bp = jnp.concatenate([W3_0[256:], jnp.zeros((5, 256), F32)], axis=0)
    out = _stage3(x2.reshape(B, Q2, 256), pos2p.reshape(B, Q2, 8),
                  W3a, W3bp, b3_0, W3_1, b3_1, W3_2, b3_2, Wfc, bfc)
    return out


# speedup vs baseline: 1.6053x; 1.0109x over previous
import functools

import jax
import jax.numpy as jnp
from jax import lax
from jax.experimental import pallas as pl
from jax.experimental.pallas import tpu as pltpu
from jax.experimental.pallas import tpu_sc as plsc

F32 = jnp.float32
I32 = jnp.int32
K = 64


def _fps_body(px_ref, py_ref, pz_ref, ox_ref, oy_ref, oz_ref, *, M):
    B, N = px_ref.shape
    px = px_ref[...]
    py = py_ref[...]
    pz = pz_ref[...]
    iota_n = lax.broadcasted_iota(I32, (B, N), 1)
    iota_m = lax.broadcasted_iota(I32, (B, M), 1)

    def body(i, st):
        dists, sx, sy, sz = st
        m = jnp.max(dists, axis=1, keepdims=True)
        nxt = jnp.min(jnp.where(dists == m, iota_n, N), axis=1,
                      keepdims=True)
        selm = iota_n == nxt
        qx = jnp.sum(jnp.where(selm, px, 0.0), axis=1, keepdims=True)
        qy = jnp.sum(jnp.where(selm, py, 0.0), axis=1, keepdims=True)
        qz = jnp.sum(jnp.where(selm, pz, 0.0), axis=1, keepdims=True)
        ddx = px - qx
        ddy = py - qy
        ddz = pz - qz
        d = (ddx * ddx + ddy * ddy) + ddz * ddz
        put = iota_m == i
        return (jnp.minimum(dists, d), jnp.where(put, qx, sx),
                jnp.where(put, qy, sy), jnp.where(put, qz, sz))

    z = jnp.zeros((B, M), F32)
    _, sx, sy, sz = lax.fori_loop(
        0, M, body, (jnp.full((B, N), 3.0e38, F32), z, z, z))
    ox_ref[...] = sx
    oy_ref[...] = sy
    oz_ref[...] = sz


def _fps(px, py, pz, M):
    B = px.shape[0]
    out = jax.ShapeDtypeStruct((B, M), F32)
    return pl.pallas_call(
        functools.partial(_fps_body, M=M),
        out_shape=[out, out, out],
    )(px, py, pz)


def _iota16():
    return lax.broadcasted_iota(I32, (16,), 0)


def _splat(val, dtype=I32):
    return jnp.full((16,), val, dtype)


def _sc_search_gather_sa1(px, py, pz, qx, qy, qz, r2):
    B, N = px.shape
    BQ = qx.shape[0]
    Q = BQ // B
    NW = 32
    QPW = BQ // NW
    WPB = NW // B
    mesh = plsc.VectorSubcoreMesh(core_axis_name="c", subcore_axis_name="s")

    @functools.partial(
        pl.kernel,
        mesh=mesh,
        compiler_params=pltpu.CompilerParams(needs_layout_passes=False),
        out_type=[
            jax.ShapeDtypeStruct((BQ * K * 8,), F32),
            jax.ShapeDtypeStruct((BQ,), I32),
        ],
        scratch_types=[
            pltpu.VMEM((N,), F32),
            pltpu.VMEM((N,), F32),
            pltpu.VMEM((N,), F32),
            pltpu.VMEM((QPW,), F32),
            pltpu.VMEM((QPW,), F32),
            pltpu.VMEM((QPW,), F32),
            *([pltpu.VMEM((K * 8,), F32)] * 8),
            pltpu.VMEM((QPW,), I32),
        ],
    )
    def body(px_h, py_h, pz_h, qx_h, qy_h, qz_h, feat_h, cnt_h,
             pxs, pys, pzs, qxs, qys, qzs,
             ot0, ot1, ot2, ot3, ot4, ot5, ot6, ot7, cnts):
        wid = lax.axis_index("s") * 2 + lax.axis_index("c")
        b = wid // WPB
        qbase = wid * QPW
        pltpu.sync_copy(px_h.at[b], pxs)
        pltpu.sync_copy(py_h.at[b], pys)
        pltpu.sync_copy(pz_h.at[b], pzs)
        pltpu.sync_copy(qx_h.at[pl.ds(qbase, QPW)], qxs)
        pltpu.sync_copy(qy_h.at[pl.ds(qbase, QPW)], qys)
        pltpu.sync_copy(qz_h.at[pl.ds(qbase, QPW)], qzs)

        zf = jnp.zeros((16,), F32)
        ots = [ot0, ot1, ot2, ot3, ot4, ot5, ot6, ot7]
        for ot in ots:
            for j in range(K * 8 // 16):
                ot[pl.ds(j * 16, 16)] = zf

        lanes = _iota16()

        def per_quad(p, _):
            q0 = p * 8
            iqs = [_splat(q0 + t) for t in range(8)]
            qxv = [plsc.load_gather(qxs, [iq]) for iq in iqs]
            qyv = [plsc.load_gather(qys, [iq]) for iq in iqs]
            qzv = [plsc.load_gather(qzs, [iq]) for iq in iqs]

            def chunk(ci, carry):
                base = ci * 16
                pxv = pxs[pl.ds(base, 16)]
                pyv = pys[pl.ds(base, 16)]
                pzv = pzs[pl.ds(base, 16)]
                ms = []
                for t in range(8):
                    dx = pxv - qxv[t]
                    dy = pyv - qyv[t]
                    dz = pzv - qzv[t]
                    d2 = (dx * dx + dy * dy) + dz * dz
                    m = d2 <= r2
                    s = carry[t] + plsc.cumsum(m.astype(I32)) - 1
                    ok = m & (s < K)
                    f = s * 8
                    plsc.store_scatter(ots[t], [f], pxv, mask=ok)
                    plsc.store_scatter(ots[t], [f + 1], pyv, mask=ok)
                    plsc.store_scatter(ots[t], [f + 2], pzv, mask=ok)
                    plsc.store_scatter(ots[t], [f + 3], dx, mask=ok)
                    plsc.store_scatter(ots[t], [f + 4], dy, mask=ok)
                    plsc.store_scatter(ots[t], [f + 5], dz, mask=ok)
                    ms.append(m)
                return tuple(
                    carry[t] + plsc.all_reduce_population_count(ms[t])
                    for t in range(8))

            z16 = jnp.zeros((16,), I32)
            cnt4 = lax.fori_loop(0, N // 16, chunk, (z16,) * 8)

            def copy_out(ot, cnt_v, row):
                cs = jnp.max(cnt_v)
                base = row * (K * 8)

                @pl.when(cs <= 16)
                def _():
                    pltpu.sync_copy(ot.at[pl.ds(0, 128)],
                                    feat_h.at[pl.ds(base, 128)])

                @pl.when((cs > 16) & (cs <= 32))
                def _():
                    pltpu.sync_copy(ot.at[pl.ds(0, 256)],
                                    feat_h.at[pl.ds(base, 256)])

                @pl.when((cs > 32) & (cs <= 48))
                def _():
                    pltpu.sync_copy(ot.at[pl.ds(0, 384)],
                                    feat_h.at[pl.ds(base, 384)])

                @pl.when(cs > 48)
                def _():
                    pltpu.sync_copy(ot, feat_h.at[pl.ds(base, K * 8)])

            for t in range(8):
                cv = jnp.minimum(cnt4[t], K)
                plsc.store_scatter(cnts, [iqs[t]], cv, mask=lanes == 0)
                copy_out(ots[t], cv, qbase + q0 + t)
            return 0

        lax.fori_loop(0, QPW // 8, per_quad, 0)
        pltpu.sync_copy(cnts, cnt_h.at[pl.ds(qbase, QPW)])

    return body(px, py, pz, qx, qy, qz)


def _sc_search_gather_sa2(px, py, pz, qx, qy, qz, t1, r2):
    B, N = px.shape
    BQ = qx.shape[0]
    NW = 32
    QPW = BQ // NW
    WPB = NW // B
    D = t1.shape[1]
    mesh = plsc.VectorSubcoreMesh(core_axis_name="c", subcore_axis_name="s")

    @functools.partial(
        pl.kernel,
        mesh=mesh,
        compiler_params=pltpu.CompilerParams(needs_layout_passes=False),
        out_type=[
            jax.ShapeDtypeStruct((BQ, K, D), F32),
            jax.ShapeDtypeStruct((BQ,), I32),
        ],
        scratch_types=[
            pltpu.VMEM((N,), F32),
            pltpu.VMEM((N,), F32),
            pltpu.VMEM((N,), F32),
            pltpu.VMEM((QPW,), F32),
            pltpu.VMEM((QPW,), F32),
            pltpu.VMEM((QPW,), F32),
            pltpu.VMEM((K,), I32),
            pltpu.VMEM((K, D), F32),
            pltpu.VMEM((QPW,), I32),
            pltpu.SemaphoreType.DMA,
        ],
    )
    def body(px_h, py_h, pz_h, qx_h, qy_h, qz_h, t1_h, g_h, cnt_h,
             pxs, pys, pzs, qxs, qys, qzs, idxb, rows, cnts, sem0):
        wid = lax.axis_index("s") * 2 + lax.axis_index("c")
        b = wid // WPB
        qbase = wid * QPW
        pbase = b * N
        pltpu.sync_copy(px_h.at[b], pxs)
        pltpu.sync_copy(py_h.at[b], pys)
        pltpu.sync_copy(pz_h.at[b], pzs)
        pltpu.sync_copy(qx_h.at[pl.ds(qbase, QPW)], qxs)
        pltpu.sync_copy(qy_h.at[pl.ds(qbase, QPW)], qys)
        pltpu.sync_copy(qz_h.at[pl.ds(qbase, QPW)], qzs)

        zi = jnp.zeros((16,), I32)
        for j in range(K // 16):
            idxb[pl.ds(j * 16, 16)] = zi + pbase

        lanes = _iota16()

        pltpu.async_copy(t1_h.at[idxb], rows, sem0).wait()

        def per_query(q, _):
            iq = _splat(q)
            qxv = plsc.load_gather(qxs, [iq])
            qyv = plsc.load_gather(qys, [iq])
            qzv = plsc.load_gather(qzs, [iq])

            def chunk(ci, cnt_v):
                base = ci * 16
                pxv = pxs[pl.ds(base, 16)]
                pyv = pys[pl.ds(base, 16)]
                pzv = pzs[pl.ds(base, 16)]
                dx = pxv - qxv
                dy = pyv - qyv
                dz = pzv - qzv
                d2 = (dx * dx + dy * dy) + dz * dz
                m = d2 <= r2
                slot = cnt_v + plsc.cumsum(m.astype(I32)) - 1
                ok = m & (slot < K)
                plsc.store_scatter(idxb, [slot],
                                   lanes + (base + pbase), mask=ok)
                return cnt_v + plsc.all_reduce_population_count(m)

            cnt_v = lax.fori_loop(0, N // 16, chunk, jnp.zeros((16,), I32))
            cnt_v = jnp.minimum(cnt_v, K)
            plsc.store_scatter(cnts, [iq], cnt_v, mask=lanes == 0)

            cnt_s = jnp.max(cnt_v)

            @pl.when(cnt_s <= 16)
            def _():
                pltpu.async_copy(t1_h.at[idxb.at[pl.ds(0, 16)]],
                                 rows.at[pl.ds(0, 16)], sem0).wait()
                pltpu.sync_copy(rows.at[pl.ds(0, 16)],
                                g_h.at[qbase + q, pl.ds(0, 16)])

            @pl.when((cnt_s > 16) & (cnt_s <= 32))
            def _():
                pltpu.async_copy(t1_h.at[idxb.at[pl.ds(0, 32)]],
                                 rows.at[pl.ds(0, 32)], sem0).wait()
                pltpu.sync_copy(rows.at[pl.ds(0, 32)],
                                g_h.at[qbase + q, pl.ds(0, 32)])

            @pl.when((cnt_s > 32) & (cnt_s <= 48))
            def _():
                pltpu.async_copy(t1_h.at[idxb.at[pl.ds(0, 48)]],
                                 rows.at[pl.ds(0, 48)], sem0).wait()
                pltpu.sync_copy(rows.at[pl.ds(0, 48)],
                                g_h.at[qbase + q, pl.ds(0, 48)])

            @pl.when(cnt_s > 48)
            def _():
                pltpu.async_copy(t1_h.at[idxb], rows, sem0).wait()
                pltpu.sync_copy(rows, g_h.at[qbase + q])

            return 0

        lax.fori_loop(0, QPW, per_query, 0)
        pltpu.sync_copy(cnts, cnt_h.at[pl.ds(qbase, QPW)])

    return body(px, py, pz, qx, qy, qz, t1)


def _mlp1_body(feat_ref, pen_ref, cnt_ref, w0_ref, b0_ref, w1_ref, b1_ref,
               w2_ref, b2_ref, pos1_ref, wx2_ref, wr2_ref, x1_ref, t1_ref):
    QB = feat_ref.shape[0] // K
    cs = jnp.max(cnt_ref[...])

    def compute(KS):
        X = feat_ref[...].reshape(QB, K, 8)[:, :KS, :].reshape(QB * KS, 8)
        pen = pen_ref[...].reshape(QB, K, 1)[:, :KS, :].reshape(QB * KS, 1)
        h = (jnp.dot(X, w0_ref[...], preferred_element_type=F32)
             + b0_ref[0:1, :])
        h = jnp.maximum(h, 0.0)
        h = (jnp.dot(h, w1_ref[...], preferred_element_type=F32)
             + b1_ref[0:1, :])
        h = jnp.maximum(h, 0.0)
        h = (jnp.dot(h, w2_ref[...], preferred_element_type=F32)
             + b2_ref[0:1, :])
        h = jnp.where(pen < 0.0, F32(-3.0e38), h)
        x1 = jnp.max(h.reshape(QB, KS, 128), axis=1)
        x1_ref[...] = x1
        t1_ref[...] = (
            jnp.dot(x1, wx2_ref[...], preferred_element_type=F32)
            + jnp.dot(pos1_ref[...], wr2_ref[...],
                      preferred_element_type=F32))

    @pl.when(cs <= 16)
    def _():
        compute(16)

    @pl.when((cs > 16) & (cs <= 32))
    def _():
        compute(32)

    @pl.when((cs > 32) & (cs <= 48))
    def _():
        compute(48)

    @pl.when(cs > 48)
    def _():
        compute(64)


def _mlp2_body(g_ref, pen_ref, cnt_ref, pos2_ref, wr2_ref, b0_ref, w1_ref,
               b1_ref, w2_ref, b2_ref, x2_ref):
    QB, _, D = g_ref.shape
    cs = jnp.max(cnt_ref[...])
    U = jnp.dot(pos2_ref[...], wr2_ref[...], preferred_element_type=F32)

    def compute(KS):
        h = (g_ref[:, :KS, :]
             - lax.broadcast_in_dim(U, (QB, KS, D), (0, 2))
             + lax.broadcast_in_dim(b0_ref[0:1, :], (QB, KS, D), (1, 2)))
        h = jnp.maximum(h, 0.0).reshape(QB * KS, D)
        h = (jnp.dot(h, w1_ref[...], preferred_element_type=F32)
             + b1_ref[0:1, :])
        h = jnp.maximum(h, 0.0)
        h = (jnp.dot(h, w2_ref[...], preferred_element_type=F32)
             + b2_ref[0:1, :])
        pen = pen_ref[...].reshape(QB, K, 1)[:, :KS, :].reshape(QB * KS, 1)
        h = jnp.where(pen < 0.0, F32(-3.0e38), h)
        x2_ref[...] = jnp.max(h.reshape(QB, KS, 256), axis=1)

    @pl.when(cs <= 16)
    def _():
        compute(16)

    @pl.when((cs > 16) & (cs <= 32))
    def _():
        compute(32)

    @pl.when((cs > 32) & (cs <= 48))
    def _():
        compute(48)

    @pl.when(cs > 48)
    def _():
        compute(64)


def _stage3_body(x2_ref, pos2_ref, w3a_ref, w3b_ref, b0_ref, w1_ref, b1_ref,
                 w2_ref, b2_ref, wfc_ref, bfc_ref, out_ref):
    x2 = x2_ref[0]
    p = pos2_ref[0]
    h = (jnp.dot(x2, w3a_ref[...], preferred_element_type=F32)
         + jnp.dot(p, w3b_ref[...], preferred_element_type=F32)
         + b0_ref[0:1, :])
    h = jnp.maximum(h, 0.0)
    h = jnp.dot(h, w1_ref[...], preferred_element_type=F32) + b1_ref[0:1, :]
    h = jnp.maximum(h, 0.0)
    h = jnp.dot(h, w2_ref[...], preferred_element_type=F32) + b2_ref[0:1, :]
    g = jnp.max(h, axis=0, keepdims=True)
    out_ref[0] = jnp.dot(g, wfc_ref[...], preferred_element_type=F32) \
        + bfc_ref[0:1, :]


def _tile8(b):
    return jnp.tile(b.reshape(1, -1), (8, 1))


def _penalty(cnt):
    BQ = cnt.shape[0]
    kio = lax.broadcasted_iota(I32, (BQ, K), 1)
    pen = jnp.where(kio < cnt[:, None], 0.0, -jnp.inf).astype(F32)
    return pen.reshape(BQ * K, 1)


def _mlp1(feat, cnt, pos1p, W0p, b0, W1, b1, W2, b2, Wx2, Wr2p):
    BQ = cnt.shape[0]
    QB = 64
    grid = BQ // QB
    feat = feat.reshape(BQ * K, 8)
    full = lambda shp: pl.BlockSpec(shp, lambda i: (0, 0))
    return pl.pallas_call(
        _mlp1_body,
        grid=(grid,),
        in_specs=[
            pl.BlockSpec((QB * K, 8), lambda i: (i, 0)),
            pl.BlockSpec((QB * K, 1), lambda i: (i, 0)),
            pl.BlockSpec((QB, 1), lambda i: (i, 0)),
            full((8, 64)), full((8, 64)),
            full((64, 64)), full((8, 64)),
            full((64, 128)), full((8, 128)),
            pl.BlockSpec((QB, 8), lambda i: (i, 0)),
            full((128, 128)), full((8, 128)),
        ],
        out_specs=[
            pl.BlockSpec((QB, 128), lambda i: (i, 0)),
            pl.BlockSpec((QB, 128), lambda i: (i, 0)),
        ],
        out_shape=[
            jax.ShapeDtypeStruct((BQ, 128), F32),
            jax.ShapeDtypeStruct((BQ, 128), F32),
        ],
    )(feat, _penalty(cnt), cnt.reshape(BQ, 1), W0p, _tile8(b0), W1,
      _tile8(b1), W2, _tile8(b2), pos1p, Wx2, Wr2p)


def _mlp2(G, cnt, pos2p, Wr2p, b0, W1, b1, W2, b2):
    BQ = G.shape[0]
    QB = 32
    grid = BQ // QB
    full = lambda shp: pl.BlockSpec(shp, lambda i: (0, 0))
    return pl.pallas_call(
        _mlp2_body,
        grid=(grid,),
        in_specs=[
            pl.BlockSpec((QB, K, 128), lambda i: (i, 0, 0)),
            pl.BlockSpec((QB * K, 1), lambda i: (i, 0)),
            pl.BlockSpec((QB, 1), lambda i: (i, 0)),
            pl.BlockSpec((QB, 8), lambda i: (i, 0)),
            full((8, 128)), full((8, 128)),
            full((128, 128)), full((8, 128)),
            full((128, 256)), full((8, 256)),
        ],
        out_specs=pl.BlockSpec((QB, 256), lambda i: (i, 0)),
        out_shape=jax.ShapeDtypeStruct((BQ, 256), F32),
    )(G, _penalty(cnt), cnt.reshape(BQ, 1), pos2p, Wr2p, _tile8(b0), W1,
      _tile8(b1), W2, _tile8(b2))


def _stage3(x2, pos2p, W3a, W3bp, b0, W1, b1, W2, b2, Wfc, bfc):
    B = x2.shape[0]
    full = lambda shp: pl.BlockSpec(shp, lambda i: (0, 0))
    out = pl.pallas_call(
        _stage3_body,
        grid=(B,),
        in_specs=[
            pl.BlockSpec((1, 512, 256), lambda i: (i, 0, 0)),
            pl.BlockSpec((1, 512, 8), lambda i: (i, 0, 0)),
            full((256, 256)), full((8, 256)), full((8, 256)),
            full((256, 512)), full((8, 512)),
            full((512, 1024)), full((8, 1024)),
            full((1024, 256)), full((8, 256)),
        ],
        out_specs=pl.BlockSpec((1, 1, 256), lambda i: (i, 0, 0)),
        out_shape=jax.ShapeDtypeStruct((B, 1, 256), F32),
    )(x2, pos2p, W3a, W3bp, _tile8(b0), W1, _tile8(b1), W2, _tile8(b2),
      Wfc, _tile8(bfc))
    return out[:, 0, :]


def kernel(data, W1_0, b1_0, W1_1, b1_1, W1_2, b1_2, W2_0, b2_0, W2_1, b2_1,
           W2_2, b2_2, W3_0, b3_0, W3_1, b3_1, W3_2, b3_2, Wfc, bfc):
    B, N, _ = data.shape
    Q1, Q2 = N // 2, N // 8

    pos_t = jnp.transpose(data, (0, 2, 1))
    p0 = [pos_t[:, c, :] for c in range(3)]
    p1 = _fps(*p0, Q1)
    p2 = _fps(*p1, Q2)

    q1 = [c.reshape(B * Q1) for c in p1]
    q2 = [c.reshape(B * Q2) for c in p2]

    feat1, cnt1 = _sc_search_gather_sa1(*p0, *q1, F32(0.2 * 0.2))

    pos1p = jnp.pad(jnp.stack(p1, axis=-1).reshape(B * Q1, 3),
                    ((0, 0), (0, 5)))
    pos2p = jnp.pad(jnp.stack(p2, axis=-1).reshape(B * Q2, 3),
                    ((0, 0), (0, 5)))
    W0p = jnp.concatenate([W1_0, jnp.zeros((2, 64), F32)], axis=0)
    Wx2 = W2_0[:128]
    Wr2p = jnp.concatenate([W2_0[128:], jnp.zeros((5, 128), F32)], axis=0)

    x1, T1 = _mlp1(feat1, cnt1, pos1p, W0p, b1_0, W1_1, b1_1, W1_2, b1_2,
                   Wx2, Wr2p)

    G, cnt2 = _sc_search_gather_sa2(*p1, *q2, T1, F32(0.4 * 0.4))
    x2 = _mlp2(G, cnt2, pos2p, Wr2p, b2_0, W2_1, b2_1, W2_2, b2_2)

    W3a = W3_0[:256]
    W3bp = jnp.concatenate([W3_0[256:], jnp.zeros((5, 256), F32)], axis=0)
    out = _stage3(x2.reshape(B, Q2, 256), pos2p.reshape(B, Q2, 8),
                  W3a, W3bp, b3_0, W3_1, b3_1, W3_2, b3_2, Wfc, bfc)
    return out
